# Initial kernel scaffold; baseline (speedup 1.0000x reference)
import jax
import jax.numpy as jnp
from jax.experimental import pallas as pl


def kernel(x, edge_index, batch, W0, b0, g0, be0, W1, b1, g1, be1, W2, b2, g2, be2, W3, b3, g3, be3, W4, b4, g4, be4, Wemb, bemb, Wm0, bm0, gm0, bem0, Wm1, bm1, gm1, bem1, Wout, bout):
    raise NotImplementedError("write your pallas kernel here")


# speedup vs baseline: 2.4982x; 2.4982x over previous
import functools

import jax
import jax.numpy as jnp
from jax import lax
from jax.experimental import pallas as pl
from jax.experimental.pallas import tpu as pltpu
from jax.experimental.pallas import tpu_sc as plsc

N = 100000
E = 1600000
NODE_DIM = 37
NUM_GRAPHS = 512

NC = 2
NS = 16
L = 16

N_PAD = 100352
STRIPE = N_PAD // NS
BN = 2048
N_BLOCKS = N_PAD // BN

K_BATCH = 8
STEP_E = K_BATCH * 128
E_PAD = 16 * STEP_E * 196
ROWS_PER_TILE = (E_PAD // NS) // 128
AGG_STEPS = ROWS_PER_TILE // K_BATCH
DEG_STEPS = (E_PAD // (NC * NS)) // STEP_E

POOL_ROWS = 640
POOL_STEP = 128
POOL_BLOCKS = N_PAD // POOL_STEP
POOL_STEPS = -(-POOL_BLOCKS // (NC * NS))

_MESH = plsc.VectorSubcoreMesh(
    core_axis_name="c", subcore_axis_name="s", num_cores=NC, num_subcores=NS)


def _pad2(a, r, c):
    return jnp.pad(a, ((0, r - a.shape[0]), (0, c - a.shape[1])))


def _row(a, c):
    return jnp.pad(a, (0, c - a.shape[0])).reshape(1, c)


def _deg_body(dst2d, zeros16, out, acc, dstb, ones_buf):
    cid = lax.axis_index("c")
    sid = lax.axis_index("s")
    one_row = jnp.where(lax.iota(jnp.int32, L) == 0, 1.0, 0.0)
    for r in range(128):
        ones_buf[r, :] = one_row
    pltpu.sync_copy(zeros16.at[pl.ds(sid * STRIPE, STRIPE)],
                    acc.at[pl.ds(sid * STRIPE, STRIPE)])
    plsc.subcore_barrier()

    base_rows = (cid * NS + sid) * (DEG_STEPS * K_BATCH)

    def step(j, carry):
        rowbase = base_rows + j * K_BATCH
        pltpu.sync_copy(dst2d.at[pl.ds(rowbase, K_BATCH)], dstb)
        for b in range(K_BATCH):
            pltpu.sync_copy(ones_buf, acc.at[dstb.at[b]], add=True)
        return carry

    lax.fori_loop(0, DEG_STEPS, step, 0)
    plsc.subcore_barrier()
    pltpu.sync_copy(acc.at[pl.ds(sid * STRIPE, STRIPE)],
                    out.at[cid, pl.ds(sid * STRIPE, STRIPE)])


def _deg_kernel(dst2d, zeros16):
    return pl.kernel(
        _deg_body,
        out_type=jax.ShapeDtypeStruct((NC, N_PAD, L), jnp.float32),
        mesh=_MESH,
        compiler_params=pltpu.CompilerParams(use_tc_tiling_on_sc=False),
        scratch_types=[
            pltpu.VMEM_SHARED((N_PAD, L), jnp.float32),
            pltpu.VMEM((K_BATCH, 128), jnp.int32),
            pltpu.VMEM((128, L), jnp.float32),
        ],
    )(dst2d, zeros16)


def _agg_body(nch, src2d, dst2d, p2d, zeros16, out, acc, srcb, dstb, gidxb,
              rows, gsem):
    cid = lax.axis_index("c")
    sid = lax.axis_index("s")
    for ci in range((nch + 1) // 2):
        chunk = cid + 2 * ci

        @pl.when(chunk < nch)
        def _chunk():
            pltpu.sync_copy(zeros16.at[pl.ds(sid * STRIPE, STRIPE)],
                            acc.at[pl.ds(sid * STRIPE, STRIPE)])
            plsc.subcore_barrier()

            def step(j, carry):
                rowbase = sid * ROWS_PER_TILE + j * K_BATCH
                pltpu.sync_copy(src2d.at[pl.ds(rowbase, K_BATCH)], srcb)
                pltpu.sync_copy(dst2d.at[pl.ds(rowbase, K_BATCH)], dstb)
                for v in range(K_BATCH):
                    for u in range(128 // L):
                        s16 = srcb[v, pl.ds(u * L, L)]
                        gidxb[v, pl.ds(u * L, L)] = s16 * nch + chunk
                descs = [
                    pltpu.async_copy(p2d.at[gidxb.at[b]],
                                     rows.at[pl.ds(b * 128, 128)], gsem)
                    for b in range(K_BATCH)
                ]
                for d in descs:
                    d.wait()
                for b in range(K_BATCH):
                    pltpu.sync_copy(rows.at[pl.ds(b * 128, 128)],
                                    acc.at[dstb.at[b]], add=True)
                return carry

            lax.fori_loop(0, AGG_STEPS, step, 0)
            plsc.subcore_barrier()
            pltpu.sync_copy(
                acc.at[pl.ds(sid * STRIPE, STRIPE)],
                out.at[chunk, pl.ds(sid * STRIPE, STRIPE)])
            plsc.subcore_barrier()


@functools.lru_cache(maxsize=None)
def _agg_kernel(nch):
    return pl.kernel(
        functools.partial(_agg_body, nch),
        out_type=jax.ShapeDtypeStruct((nch, N_PAD, L), jnp.float32),
        mesh=_MESH,
        compiler_params=pltpu.CompilerParams(use_tc_tiling_on_sc=False),
        scratch_types=[
            pltpu.VMEM_SHARED((N_PAD, L), jnp.float32),
            pltpu.VMEM((K_BATCH, 128), jnp.int32),
            pltpu.VMEM((K_BATCH, 128), jnp.int32),
            pltpu.VMEM((K_BATCH, 128), jnp.int32),
            pltpu.VMEM((STEP_E, L), jnp.float32),
            pltpu.SemaphoreType.DMA,
        ],
    )


def _pool_body(emb, batch2d, zeros_pool, out, acc, rbuf, bidx):
    cid = lax.axis_index("c")
    sid = lax.axis_index("s")

    @pl.when(sid == 0)
    def _z():
        pltpu.sync_copy(zeros_pool, acc)

    plsc.subcore_barrier()
    wid = cid * NS + sid

    def step(j, carry):
        blk = wid + (NC * NS) * j

        @pl.when(blk < POOL_BLOCKS)
        def _():
            rowbase = blk * POOL_STEP
            pltpu.sync_copy(emb.at[pl.ds(rowbase, POOL_STEP)], rbuf)
            pltpu.sync_copy(batch2d.at[pl.ds(blk, 1)], bidx)
            pltpu.sync_copy(rbuf, acc.at[bidx.at[0]], add=True)

        return carry

    lax.fori_loop(0, POOL_STEPS, step, 0)
    plsc.subcore_barrier()
    rows_per_tile = POOL_ROWS // NS
    pltpu.sync_copy(acc.at[pl.ds(sid * rows_per_tile, rows_per_tile)],
                    out.at[cid, pl.ds(sid * rows_per_tile, rows_per_tile)])


def _pool_kernel(emb, batch2d, zeros_pool):
    return pl.kernel(
        _pool_body,
        out_type=jax.ShapeDtypeStruct((NC, POOL_ROWS, 128), jnp.float32),
        mesh=_MESH,
        compiler_params=pltpu.CompilerParams(use_tc_tiling_on_sc=False),
        scratch_types=[
            pltpu.VMEM_SHARED((POOL_ROWS, 128), jnp.float32),
            pltpu.VMEM((POOL_STEP, 128), jnp.float32),
            pltpu.VMEM((1, POOL_STEP), jnp.int32),
        ],
    )(emb, batch2d, zeros_pool)


def _prep_body(deg16_ref, x_ref, dinv_ref, p0_ref):
    i = pl.program_id(0)
    deg = deg16_ref[0, :, 0:1] + deg16_ref[1, :, 0:1] + 1.0
    rows = i * BN + lax.broadcasted_iota(jnp.int32, (BN, 1), 0)
    dinv = jnp.where(rows < N, lax.rsqrt(deg), 0.0)
    dinv_ref[...] = dinv
    p0_ref[...] = x_ref[...] * dinv


def _prep_kernel(deg16, x_pad):
    din = x_pad.shape[1]
    return pl.pallas_call(
        _prep_body,
        grid=(N_BLOCKS,),
        in_specs=[
            pl.BlockSpec((NC, BN, L), lambda i: (0, i, 0)),
            pl.BlockSpec((BN, din), lambda i: (i, 0)),
        ],
        out_specs=[
            pl.BlockSpec((BN, 1), lambda i: (i, 0)),
            pl.BlockSpec((BN, din), lambda i: (i, 0)),
        ],
        out_shape=[
            jax.ShapeDtypeStruct((N_PAD, 1), jnp.float32),
            jax.ShapeDtypeStruct((N_PAD, din), jnp.float32),
        ],
    )(deg16, x_pad)


def _layer_body(final, nch, u3_ref, p_ref, dinv_ref, w_ref, b_ref, gs_ref,
                be_ref, out_ref):
    c = pl.program_id(1)
    dinv = dinv_ref[...]

    @pl.when(c == 0)
    def _init():
        out_ref[...] = jnp.dot(p_ref[...] * dinv, w_ref[...],
                               preferred_element_type=jnp.float32) + b_ref[...]

    wc = w_ref[pl.ds(c * L, L), :]
    out_ref[...] += jnp.dot(u3_ref[0] * dinv, wc,
                            preferred_element_type=jnp.float32)

    @pl.when(c == nch - 1)
    def _fin():
        h = jnp.maximum(out_ref[...], 0.0) * gs_ref[...] + be_ref[...]
        out_ref[...] = h if final else h * dinv


def _layer_kernel(u3, p, dinv, w, b, gs, be, final):
    nch = u3.shape[0]
    din = p.shape[1]
    dout = w.shape[1]
    return pl.pallas_call(
        functools.partial(_layer_body, final, nch),
        grid=(N_BLOCKS, nch),
        in_specs=[
            pl.BlockSpec((1, BN, L), lambda i, c: (c, i, 0)),
            pl.BlockSpec((BN, din), lambda i, c: (i, 0)),
            pl.BlockSpec((BN, 1), lambda i, c: (i, 0)),
            pl.BlockSpec((din, dout), lambda i, c: (0, 0)),
            pl.BlockSpec((1, dout), lambda i, c: (0, 0)),
            pl.BlockSpec((1, dout), lambda i, c: (0, 0)),
            pl.BlockSpec((1, dout), lambda i, c: (0, 0)),
        ],
        out_specs=pl.BlockSpec((BN, dout), lambda i, c: (i, 0)),
        out_shape=jax.ShapeDtypeStruct((N_PAD, dout), jnp.float32),
    )(u3, p, dinv, w, b, gs, be)


def _emb_body(h_ref, w_ref, b_ref, out_ref):
    z = jnp.dot(h_ref[...], w_ref[...],
                preferred_element_type=jnp.float32) + b_ref[...]
    m = jnp.max(z, axis=-1, keepdims=True)
    e = jnp.exp(z - m)
    out_ref[...] = e / jnp.sum(e, axis=-1, keepdims=True)


def _emb_kernel(h5, w, b):
    din = h5.shape[1]
    return pl.pallas_call(
        _emb_body,
        grid=(N_BLOCKS,),
        in_specs=[
            pl.BlockSpec((BN, din), lambda i: (i, 0)),
            pl.BlockSpec((din, 128), lambda i: (0, 0)),
            pl.BlockSpec((1, 128), lambda i: (0, 0)),
        ],
        out_specs=pl.BlockSpec((BN, 128), lambda i: (i, 0)),
        out_shape=jax.ShapeDtypeStruct((N_PAD, 128), jnp.float32),
    )(h5, w, b)


def _head_body(pool_ref, wm0, bm0, gsm0, bem0, wm1, bm1, gsm1, bem1, wo, bo,
               out_ref):
    hg = pool_ref[0, :NUM_GRAPHS, :] + pool_ref[1, :NUM_GRAPHS, :]
    z1 = jnp.dot(hg, wm0[...], preferred_element_type=jnp.float32) + bm0[...]
    h1 = jnp.maximum(z1, 0.0) * gsm0[...] + bem0[...]
    z2 = jnp.dot(h1, wm1[...], preferred_element_type=jnp.float32) + bm1[...]
    h2 = jnp.maximum(z2, 0.0) * gsm1[...] + bem1[...]
    out_ref[...] = jnp.dot(h2, wo[...],
                           preferred_element_type=jnp.float32) + bo[...]


def _head_kernel(pool, wm0, bm0, gsm0, bem0, wm1, bm1, gsm1, bem1, wo, bo):
    full = lambda a: pl.BlockSpec(a.shape, lambda: tuple(0 for _ in a.shape))
    return pl.pallas_call(
        _head_body,
        in_specs=[full(pool), full(wm0), full(bm0), full(gsm0), full(bem0),
                  full(wm1), full(bm1), full(gsm1), full(bem1), full(wo),
                  full(bo)],
        out_specs=pl.BlockSpec((NUM_GRAPHS, 128), lambda: (0, 0)),
        out_shape=jax.ShapeDtypeStruct((NUM_GRAPHS, 128), jnp.float32),
    )(pool, wm0, bm0, gsm0, bem0, wm1, bm1, gsm1, bem1, wo, bo)


_BN_SCALE = 1.0 / (1.0 + 1e-5) ** 0.5
_CONV_PADS = [(48, 64), (64, 80), (80, 112), (112, 128), (128, 160)]


def kernel(x, edge_index, batch, W0, b0, g0, be0, W1, b1, g1, be1, W2, b2, g2,
           be2, W3, b3, g3, be3, W4, b4, g4, be4, Wemb, bemb, Wm0, bm0, gm0,
           bem0, Wm1, bm1, gm1, bem1, Wout, bout):
    f32 = jnp.float32
    x_pad = _pad2(x, N_PAD, 48)
    pad_e = E_PAD - E
    padidx = (N + (jnp.arange(pad_e, dtype=jnp.int32) % 16)).astype(jnp.int32)
    src2d = jnp.concatenate([edge_index[0], padidx]).reshape(E_PAD // 128, 128)
    dst2d = jnp.concatenate([edge_index[1], padidx]).reshape(E_PAD // 128, 128)
    batch2d = jnp.pad(batch, (0, N_PAD - N),
                      constant_values=NUM_GRAPHS).reshape(POOL_BLOCKS,
                                                          POOL_STEP)
    zeros16 = jnp.zeros((N_PAD, L), f32)
    zeros_pool = jnp.zeros((POOL_ROWS, 128), f32)

    convs = [(W0, b0, g0, be0), (W1, b1, g1, be1), (W2, b2, g2, be2),
             (W3, b3, g3, be3), (W4, b4, g4, be4)]

    deg16 = _deg_kernel(dst2d, zeros16)
    dinv, p = _prep_kernel(deg16, x_pad)

    for li, ((din_p, dout_p), (W, b, g, be)) in enumerate(zip(_CONV_PADS,
                                                              convs)):
        nch = din_p // L
        u = _agg_kernel(nch)(src2d, dst2d, p.reshape(N_PAD * nch, L),
                             zeros16)
        p = _layer_kernel(
            u, p, dinv,
            _pad2(W, din_p, dout_p), _row(b, dout_p),
            _row(g * _BN_SCALE, dout_p), _row(be, dout_p),
            final=(li == len(convs) - 1))

    wemb = _pad2(Wemb, 160, 128)
    bemb_p = jnp.full((1, 128), -1e30, f32).at[0, :100].set(bemb)
    emb = _emb_kernel(p, wemb, bemb_p)

    pool = _pool_kernel(emb, batch2d, zeros_pool)

    out_pad = _head_kernel(
        pool,
        _pad2(Wm0, 128, 64), _row(bm0, 64), _row(gm0 * _BN_SCALE, 64),
        _row(bem0, 64),
        _pad2(Wm1, 64, 32), _row(bm1, 32), _row(gm1 * _BN_SCALE, 32),
        _row(bem1, 32),
        _pad2(Wout, 32, 128), _row(bout, 128))
    return out_pad[:, :4]


# speedup vs baseline: 9.8294x; 3.9345x over previous
import functools

import jax
import jax.numpy as jnp
from jax import lax
from jax.experimental import pallas as pl
from jax.experimental.pallas import tpu as pltpu
from jax.experimental.pallas import tpu_sc as plsc

N = 100000
E = 1600000
NODE_DIM = 37
NUM_GRAPHS = 512

NC = 2
NS = 16
L = 16

N_PAD = 100352
STRIPE = N_PAD // NS
BN = 2048
N_BLOCKS = N_PAD // BN

K_BATCH = 4
STEP_E = K_BATCH * 128
E_PAD = 1605632
ROWS_PER_TILE = (E_PAD // NS) // 128
AGG_STEPS = ROWS_PER_TILE // K_BATCH
DEG_STEPS = (E_PAD // (NC * NS)) // STEP_E

POOL_ROWS = 640
POOL_STEP = 128
POOL_BLOCKS = N_PAD // POOL_STEP
POOL_STEPS = -(-POOL_BLOCKS // (NC * NS))

_MESH = plsc.VectorSubcoreMesh(
    core_axis_name="c", subcore_axis_name="s", num_cores=NC, num_subcores=NS)


def _pad2(a, r, c):
    return jnp.pad(a, ((0, r - a.shape[0]), (0, c - a.shape[1])))


def _row(a, c):
    return jnp.pad(a, (0, c - a.shape[0])).reshape(1, c)


def _deg_body(dst2d, zeros16, out, acc, dstb, ones_buf):
    cid = lax.axis_index("c")
    sid = lax.axis_index("s")
    one_row = jnp.where(lax.iota(jnp.int32, L) == 0, 1.0, 0.0)
    for r in range(128):
        ones_buf[r, :] = one_row
    pltpu.sync_copy(zeros16.at[pl.ds(sid * STRIPE, STRIPE)],
                    acc.at[pl.ds(sid * STRIPE, STRIPE)])
    plsc.subcore_barrier()

    base_rows = (cid * NS + sid) * (DEG_STEPS * K_BATCH)

    def step(j, carry):
        rowbase = base_rows + j * K_BATCH
        pltpu.sync_copy(dst2d.at[pl.ds(rowbase, K_BATCH)], dstb)
        for b in range(K_BATCH):
            pltpu.sync_copy(ones_buf, acc.at[dstb.at[b]], add=True)
        return carry

    lax.fori_loop(0, DEG_STEPS, step, 0)
    plsc.subcore_barrier()
    pltpu.sync_copy(acc.at[pl.ds(sid * STRIPE, STRIPE)],
                    out.at[cid, pl.ds(sid * STRIPE, STRIPE)])


def _deg_kernel(dst2d, zeros16):
    return pl.kernel(
        _deg_body,
        out_type=jax.ShapeDtypeStruct((NC, N_PAD, L), jnp.float32),
        mesh=_MESH,
        compiler_params=pltpu.CompilerParams(use_tc_tiling_on_sc=False),
        scratch_types=[
            pltpu.VMEM_SHARED((N_PAD, L), jnp.float32),
            pltpu.VMEM((K_BATCH, 128), jnp.int32),
            pltpu.VMEM((128, L), jnp.float32),
        ],
    )(dst2d, zeros16)


def _agg_body(nch, src2d, dst2d, p2d, zeros16, out, acc, srcb, dstb, gidxb,
              rows, isem, gsem, ssem):
    cid = lax.axis_index("c")
    sid = lax.axis_index("s")

    def idx_rowbase(step):
        return sid * ROWS_PER_TILE + step * K_BATCH

    def issue_idx(step, slot):
        base = idx_rowbase(step)
        pltpu.async_copy(src2d.at[pl.ds(base, K_BATCH)],
                         srcb.at[pl.ds(slot * K_BATCH, K_BATCH)], isem)
        pltpu.async_copy(dst2d.at[pl.ds(base, K_BATCH)],
                         dstb.at[pl.ds(slot * K_BATCH, K_BATCH)], isem)

    def wait_idx(slot):
        for ref in (srcb, dstb):
            pltpu.make_async_copy(
                src2d.at[pl.ds(0, K_BATCH)],
                ref.at[pl.ds(slot * K_BATCH, K_BATCH)], isem).wait()

    def compute_gidx(chunk, slot):
        for v in range(K_BATCH):
            r = slot * K_BATCH + v
            for u in range(128 // L):
                s16 = srcb[r, pl.ds(u * L, L)]
                gidxb[r, pl.ds(u * L, L)] = s16 * nch + chunk

    def fire_gather(slot):
        for b in range(K_BATCH):
            pltpu.async_copy(
                p2d.at[gidxb.at[slot * K_BATCH + b]],
                rows.at[pl.ds((slot * K_BATCH + b) * 128, 128)], gsem)

    def drain_gather(slot):
        for b in range(K_BATCH):
            pltpu.make_async_copy(
                p2d.at[pl.ds(0, 128)],
                rows.at[pl.ds((slot * K_BATCH + b) * 128, 128)], gsem).wait()

    def fire_scatter(slot):
        for b in range(K_BATCH):
            pltpu.async_copy(
                rows.at[pl.ds((slot * K_BATCH + b) * 128, 128)],
                acc.at[dstb.at[slot * K_BATCH + b]], ssem, add=True)

    def drain_scatter(slot):
        for b in range(K_BATCH):
            pltpu.make_async_copy(
                rows.at[pl.ds((slot * K_BATCH + b) * 128, 128)],
                acc.at[pl.ds(0, 128)], ssem).wait()

    for ci in range((nch + 1) // 2):
        chunk = cid + 2 * ci

        @pl.when(chunk < nch)
        def _chunk():
            pltpu.sync_copy(zeros16.at[pl.ds(sid * STRIPE, STRIPE)],
                            acc.at[pl.ds(sid * STRIPE, STRIPE)])
            plsc.subcore_barrier()

            pltpu.sync_copy(src2d.at[pl.ds(idx_rowbase(0), K_BATCH)],
                            srcb.at[pl.ds(0, K_BATCH)])
            compute_gidx(chunk, 0)
            fire_gather(0)
            pltpu.sync_copy(dst2d.at[pl.ds(idx_rowbase(0), K_BATCH)],
                            dstb.at[pl.ds(0, K_BATCH)])

            def step2(j2, carry):
                s0 = 2 * j2
                @pl.when(j2 > 0)
                def _():
                    drain_scatter(1)
                issue_idx(s0 + 1, 1)
                drain_gather(0)
                fire_scatter(0)
                wait_idx(1)
                compute_gidx(chunk, 1)
                fire_gather(1)
                drain_scatter(0)

                @pl.when(j2 < AGG_STEPS // 2 - 1)
                def _():
                    issue_idx(s0 + 2, 0)
                drain_gather(1)
                fire_scatter(1)

                @pl.when(j2 < AGG_STEPS // 2 - 1)
                def _():
                    wait_idx(0)
                    compute_gidx(chunk, 0)
                    fire_gather(0)
                return carry

            lax.fori_loop(0, AGG_STEPS // 2, step2, 0)
            drain_scatter(1)
            plsc.subcore_barrier()
            pltpu.sync_copy(
                acc.at[pl.ds(sid * STRIPE, STRIPE)],
                out.at[chunk, pl.ds(sid * STRIPE, STRIPE)])
            plsc.subcore_barrier()


@functools.lru_cache(maxsize=None)
def _agg_kernel(nch):
    return pl.kernel(
        functools.partial(_agg_body, nch),
        out_type=jax.ShapeDtypeStruct((nch, N_PAD, L), jnp.float32),
        mesh=_MESH,
        compiler_params=pltpu.CompilerParams(use_tc_tiling_on_sc=False),
        scratch_types=[
            pltpu.VMEM_SHARED((N_PAD, L), jnp.float32),
            pltpu.VMEM((2 * K_BATCH, 128), jnp.int32),
            pltpu.VMEM((2 * K_BATCH, 128), jnp.int32),
            pltpu.VMEM((2 * K_BATCH, 128), jnp.int32),
            pltpu.VMEM((2 * STEP_E, L), jnp.float32),
            pltpu.SemaphoreType.DMA,
            pltpu.SemaphoreType.DMA,
            pltpu.SemaphoreType.DMA,
        ],
    )


def _pool_body(emb, batch2d, zeros_pool, out, acc, rbuf, bidx):
    cid = lax.axis_index("c")
    sid = lax.axis_index("s")

    @pl.when(sid == 0)
    def _z():
        pltpu.sync_copy(zeros_pool, acc)

    plsc.subcore_barrier()
    wid = cid * NS + sid

    def step(j, carry):
        blk = wid + (NC * NS) * j

        @pl.when(blk < POOL_BLOCKS)
        def _():
            rowbase = blk * POOL_STEP
            pltpu.sync_copy(emb.at[pl.ds(rowbase, POOL_STEP)], rbuf)
            pltpu.sync_copy(batch2d.at[pl.ds(blk, 1)], bidx)
            pltpu.sync_copy(rbuf, acc.at[bidx.at[0]], add=True)

        return carry

    lax.fori_loop(0, POOL_STEPS, step, 0)
    plsc.subcore_barrier()
    rows_per_tile = POOL_ROWS // NS
    pltpu.sync_copy(acc.at[pl.ds(sid * rows_per_tile, rows_per_tile)],
                    out.at[cid, pl.ds(sid * rows_per_tile, rows_per_tile)])


def _pool_kernel(emb, batch2d, zeros_pool):
    return pl.kernel(
        _pool_body,
        out_type=jax.ShapeDtypeStruct((NC, POOL_ROWS, 128), jnp.float32),
        mesh=_MESH,
        compiler_params=pltpu.CompilerParams(use_tc_tiling_on_sc=False),
        scratch_types=[
            pltpu.VMEM_SHARED((POOL_ROWS, 128), jnp.float32),
            pltpu.VMEM((POOL_STEP, 128), jnp.float32),
            pltpu.VMEM((1, POOL_STEP), jnp.int32),
        ],
    )(emb, batch2d, zeros_pool)


def _prep_body(deg16_ref, x_ref, dinv_ref, p0_ref):
    i = pl.program_id(0)
    deg = deg16_ref[0, :, 0:1] + deg16_ref[1, :, 0:1] + 1.0
    rows = i * BN + lax.broadcasted_iota(jnp.int32, (BN, 1), 0)
    dinv = jnp.where(rows < N, lax.rsqrt(deg), 0.0)
    dinv_ref[...] = dinv
    p0_ref[...] = x_ref[...] * dinv


def _prep_kernel(deg16, x_pad):
    din = x_pad.shape[1]
    return pl.pallas_call(
        _prep_body,
        grid=(N_BLOCKS,),
        in_specs=[
            pl.BlockSpec((NC, BN, L), lambda i: (0, i, 0)),
            pl.BlockSpec((BN, din), lambda i: (i, 0)),
        ],
        out_specs=[
            pl.BlockSpec((BN, 1), lambda i: (i, 0)),
            pl.BlockSpec((BN, din), lambda i: (i, 0)),
        ],
        out_shape=[
            jax.ShapeDtypeStruct((N_PAD, 1), jnp.float32),
            jax.ShapeDtypeStruct((N_PAD, din), jnp.float32),
        ],
    )(deg16, x_pad)


def _layer_body(final, nch, u3_ref, p_ref, dinv_ref, w_ref, b_ref, gs_ref,
                be_ref, out_ref):
    c = pl.program_id(1)
    dinv = dinv_ref[...]

    @pl.when(c == 0)
    def _init():
        out_ref[...] = jnp.dot(p_ref[...] * dinv, w_ref[...],
                               preferred_element_type=jnp.float32) + b_ref[...]

    wc = w_ref[pl.ds(c * L, L), :]
    out_ref[...] += jnp.dot(u3_ref[0] * dinv, wc,
                            preferred_element_type=jnp.float32)

    @pl.when(c == nch - 1)
    def _fin():
        h = jnp.maximum(out_ref[...], 0.0) * gs_ref[...] + be_ref[...]
        out_ref[...] = h if final else h * dinv


def _layer_kernel(u3, p, dinv, w, b, gs, be, final):
    nch = u3.shape[0]
    din = p.shape[1]
    dout = w.shape[1]
    return pl.pallas_call(
        functools.partial(_layer_body, final, nch),
        grid=(N_BLOCKS, nch),
        in_specs=[
            pl.BlockSpec((1, BN, L), lambda i, c: (c, i, 0)),
            pl.BlockSpec((BN, din), lambda i, c: (i, 0)),
            pl.BlockSpec((BN, 1), lambda i, c: (i, 0)),
            pl.BlockSpec((din, dout), lambda i, c: (0, 0)),
            pl.BlockSpec((1, dout), lambda i, c: (0, 0)),
            pl.BlockSpec((1, dout), lambda i, c: (0, 0)),
            pl.BlockSpec((1, dout), lambda i, c: (0, 0)),
        ],
        out_specs=pl.BlockSpec((BN, dout), lambda i, c: (i, 0)),
        out_shape=jax.ShapeDtypeStruct((N_PAD, dout), jnp.float32),
    )(u3, p, dinv, w, b, gs, be)


def _emb_body(h_ref, w_ref, b_ref, out_ref):
    z = jnp.dot(h_ref[...], w_ref[...],
                preferred_element_type=jnp.float32) + b_ref[...]
    m = jnp.max(z, axis=-1, keepdims=True)
    e = jnp.exp(z - m)
    out_ref[...] = e / jnp.sum(e, axis=-1, keepdims=True)


def _emb_kernel(h5, w, b):
    din = h5.shape[1]
    return pl.pallas_call(
        _emb_body,
        grid=(N_BLOCKS,),
        in_specs=[
            pl.BlockSpec((BN, din), lambda i: (i, 0)),
            pl.BlockSpec((din, 128), lambda i: (0, 0)),
            pl.BlockSpec((1, 128), lambda i: (0, 0)),
        ],
        out_specs=pl.BlockSpec((BN, 128), lambda i: (i, 0)),
        out_shape=jax.ShapeDtypeStruct((N_PAD, 128), jnp.float32),
    )(h5, w, b)


def _head_body(pool_ref, wm0, bm0, gsm0, bem0, wm1, bm1, gsm1, bem1, wo, bo,
               out_ref):
    hg = pool_ref[0, :NUM_GRAPHS, :] + pool_ref[1, :NUM_GRAPHS, :]
    z1 = jnp.dot(hg, wm0[...], preferred_element_type=jnp.float32) + bm0[...]
    h1 = jnp.maximum(z1, 0.0) * gsm0[...] + bem0[...]
    z2 = jnp.dot(h1, wm1[...], preferred_element_type=jnp.float32) + bm1[...]
    h2 = jnp.maximum(z2, 0.0) * gsm1[...] + bem1[...]
    out_ref[...] = jnp.dot(h2, wo[...],
                           preferred_element_type=jnp.float32) + bo[...]


def _head_kernel(pool, wm0, bm0, gsm0, bem0, wm1, bm1, gsm1, bem1, wo, bo):
    full = lambda a: pl.BlockSpec(a.shape, lambda: tuple(0 for _ in a.shape))
    return pl.pallas_call(
        _head_body,
        in_specs=[full(pool), full(wm0), full(bm0), full(gsm0), full(bem0),
                  full(wm1), full(bm1), full(gsm1), full(bem1), full(wo),
                  full(bo)],
        out_specs=pl.BlockSpec((NUM_GRAPHS, 128), lambda: (0, 0)),
        out_shape=jax.ShapeDtypeStruct((NUM_GRAPHS, 128), jnp.float32),
    )(pool, wm0, bm0, gsm0, bem0, wm1, bm1, gsm1, bem1, wo, bo)


_BN_SCALE = 1.0 / (1.0 + 1e-5) ** 0.5
_CONV_PADS = [(48, 64), (64, 80), (80, 112), (112, 128), (128, 160)]


def kernel(x, edge_index, batch, W0, b0, g0, be0, W1, b1, g1, be1, W2, b2, g2,
           be2, W3, b3, g3, be3, W4, b4, g4, be4, Wemb, bemb, Wm0, bm0, gm0,
           bem0, Wm1, bm1, gm1, bem1, Wout, bout):
    f32 = jnp.float32
    x_pad = _pad2(x, N_PAD, 48)
    pad_e = E_PAD - E
    padidx = (N + (jnp.arange(pad_e, dtype=jnp.int32) % 16)).astype(jnp.int32)
    src2d = jnp.concatenate([edge_index[0], padidx]).reshape(E_PAD // 128, 128)
    dst2d = jnp.concatenate([edge_index[1], padidx]).reshape(E_PAD // 128, 128)
    batch2d = jnp.pad(batch, (0, N_PAD - N),
                      constant_values=NUM_GRAPHS).reshape(POOL_BLOCKS,
                                                          POOL_STEP)
    zeros16 = jnp.zeros((N_PAD, L), f32)
    zeros_pool = jnp.zeros((POOL_ROWS, 128), f32)

    convs = [(W0, b0, g0, be0), (W1, b1, g1, be1), (W2, b2, g2, be2),
             (W3, b3, g3, be3), (W4, b4, g4, be4)]

    deg16 = _deg_kernel(dst2d, zeros16)
    dinv, p = _prep_kernel(deg16, x_pad)

    for li, ((din_p, dout_p), (W, b, g, be)) in enumerate(zip(_CONV_PADS,
                                                              convs)):
        nch = din_p // L
        u = _agg_kernel(nch)(src2d, dst2d, p.reshape(N_PAD * nch, L),
                             zeros16)
        p = _layer_kernel(
            u, p, dinv,
            _pad2(W, din_p, dout_p), _row(b, dout_p),
            _row(g * _BN_SCALE, dout_p), _row(be, dout_p),
            final=(li == len(convs) - 1))

    wemb = _pad2(Wemb, 160, 128)
    bemb_p = jnp.full((1, 128), -1e30, f32).at[0, :100].set(bemb)
    emb = _emb_kernel(p, wemb, bemb_p)

    pool = _pool_kernel(emb, batch2d, zeros_pool)

    out_pad = _head_kernel(
        pool,
        _pad2(Wm0, 128, 64), _row(bm0, 64), _row(gm0 * _BN_SCALE, 64),
        _row(bem0, 64),
        _pad2(Wm1, 64, 32), _row(bm1, 32), _row(gm1 * _BN_SCALE, 32),
        _row(bem1, 32),
        _pad2(Wout, 32, 128), _row(bout, 128))
    return out_pad[:, :4]


# speedup vs baseline: 9.9157x; 1.0088x over previous
import functools

import jax
import jax.numpy as jnp
from jax import lax
from jax.experimental import pallas as pl
from jax.experimental.pallas import tpu as pltpu
from jax.experimental.pallas import tpu_sc as plsc

N = 100000
E = 1600000
NODE_DIM = 37
NUM_GRAPHS = 512

NC = 2
NS = 16
L = 16

N_PAD = 100352
STRIPE = N_PAD // NS
BN = 2048
N_BLOCKS = N_PAD // BN

K_BATCH = 4
STEP_E = K_BATCH * 128
E_PAD = 1605632
ROWS_PER_TILE = (E_PAD // NS) // 128
AGG_STEPS = ROWS_PER_TILE // K_BATCH
DEG_STEPS = (E_PAD // (NC * NS)) // STEP_E

POOL_ROWS = 640
POOL_STEP = 128
POOL_BLOCKS = N_PAD // POOL_STEP
POOL_STEPS = -(-POOL_BLOCKS // (NC * NS))

_MESH = plsc.VectorSubcoreMesh(
    core_axis_name="c", subcore_axis_name="s", num_cores=NC, num_subcores=NS)


def _pad2(a, r, c):
    return jnp.pad(a, ((0, r - a.shape[0]), (0, c - a.shape[1])))


def _row(a, c):
    return jnp.pad(a, (0, c - a.shape[0])).reshape(1, c)


def _deg_body(dst2d, zeros16, out, acc, dstb, ones_buf, isem, ssem):
    cid = lax.axis_index("c")
    sid = lax.axis_index("s")
    one_row = jnp.where(lax.iota(jnp.int32, L) == 0, 1.0, 0.0)
    for r in range(128):
        ones_buf[r, :] = one_row
    pltpu.sync_copy(zeros16.at[pl.ds(sid * STRIPE, STRIPE)],
                    acc.at[pl.ds(sid * STRIPE, STRIPE)])
    plsc.subcore_barrier()

    base_rows = (cid * NS + sid) * (DEG_STEPS * K_BATCH)

    def issue_idx(step, slot):
        pltpu.async_copy(dst2d.at[pl.ds(base_rows + step * K_BATCH, K_BATCH)],
                         dstb.at[pl.ds(slot * K_BATCH, K_BATCH)], isem)

    def wait_idx(slot):
        pltpu.make_async_copy(dst2d.at[pl.ds(0, K_BATCH)],
                              dstb.at[pl.ds(slot * K_BATCH, K_BATCH)],
                              isem).wait()

    def fire_scatter(slot):
        for b in range(K_BATCH):
            pltpu.async_copy(ones_buf, acc.at[dstb.at[slot * K_BATCH + b]],
                             ssem, add=True)

    def drain_scatter():
        for b in range(K_BATCH):
            pltpu.make_async_copy(ones_buf, acc.at[pl.ds(0, 128)],
                                  ssem).wait()

    issue_idx(0, 0)

    def step2(j2, carry):
        s0 = 2 * j2
        @pl.when(j2 > 0)
        def _():
            drain_scatter()
        issue_idx(s0 + 1, 1)
        wait_idx(0)
        fire_scatter(0)
        drain_scatter()

        @pl.when(j2 < DEG_STEPS // 2 - 1)
        def _():
            issue_idx(s0 + 2, 0)
        wait_idx(1)
        fire_scatter(1)
        return carry

    lax.fori_loop(0, DEG_STEPS // 2, step2, 0)
    drain_scatter()
    plsc.subcore_barrier()
    pltpu.sync_copy(acc.at[pl.ds(sid * STRIPE, STRIPE)],
                    out.at[cid, pl.ds(sid * STRIPE, STRIPE)])


def _deg_kernel(dst2d, zeros16):
    return pl.kernel(
        _deg_body,
        out_type=jax.ShapeDtypeStruct((NC, N_PAD, L), jnp.float32),
        mesh=_MESH,
        compiler_params=pltpu.CompilerParams(use_tc_tiling_on_sc=False),
        scratch_types=[
            pltpu.VMEM_SHARED((N_PAD, L), jnp.float32),
            pltpu.VMEM((2 * K_BATCH, 128), jnp.int32),
            pltpu.VMEM((128, L), jnp.float32),
            pltpu.SemaphoreType.DMA,
            pltpu.SemaphoreType.DMA,
        ],
    )(dst2d, zeros16)


def _agg_body(nch, src2d, dst2d, p2d, zeros16, out, acc, srcb, dstb, gidxb,
              rows, isem, gsem, ssem):
    cid = lax.axis_index("c")
    sid = lax.axis_index("s")

    def idx_rowbase(step):
        return sid * ROWS_PER_TILE + step * K_BATCH

    def issue_idx(step, slot):
        base = idx_rowbase(step)
        pltpu.async_copy(src2d.at[pl.ds(base, K_BATCH)],
                         srcb.at[pl.ds(slot * K_BATCH, K_BATCH)], isem)
        pltpu.async_copy(dst2d.at[pl.ds(base, K_BATCH)],
                         dstb.at[pl.ds(slot * K_BATCH, K_BATCH)], isem)

    def wait_idx(slot):
        for ref in (srcb, dstb):
            pltpu.make_async_copy(
                src2d.at[pl.ds(0, K_BATCH)],
                ref.at[pl.ds(slot * K_BATCH, K_BATCH)], isem).wait()

    def compute_gidx(chunk, slot):
        for v in range(K_BATCH):
            r = slot * K_BATCH + v
            for u in range(128 // L):
                s16 = srcb[r, pl.ds(u * L, L)]
                gidxb[r, pl.ds(u * L, L)] = s16 * nch + chunk

    def fire_gather(slot):
        for b in range(K_BATCH):
            pltpu.async_copy(
                p2d.at[gidxb.at[slot * K_BATCH + b]],
                rows.at[pl.ds((slot * K_BATCH + b) * 128, 128)], gsem)

    def drain_gather(slot):
        for b in range(K_BATCH):
            pltpu.make_async_copy(
                p2d.at[pl.ds(0, 128)],
                rows.at[pl.ds((slot * K_BATCH + b) * 128, 128)], gsem).wait()

    def fire_scatter(slot):
        for b in range(K_BATCH):
            pltpu.async_copy(
                rows.at[pl.ds((slot * K_BATCH + b) * 128, 128)],
                acc.at[dstb.at[slot * K_BATCH + b]], ssem, add=True)

    def drain_scatter(slot):
        for b in range(K_BATCH):
            pltpu.make_async_copy(
                rows.at[pl.ds((slot * K_BATCH + b) * 128, 128)],
                acc.at[pl.ds(0, 128)], ssem).wait()

    def chunk_body(ci, carry):
        chunk = cid + 2 * ci

        @pl.when(chunk < nch)
        def _chunk():
            pltpu.sync_copy(zeros16.at[pl.ds(sid * STRIPE, STRIPE)],
                            acc.at[pl.ds(sid * STRIPE, STRIPE)])
            plsc.subcore_barrier()

            pltpu.sync_copy(src2d.at[pl.ds(idx_rowbase(0), K_BATCH)],
                            srcb.at[pl.ds(0, K_BATCH)])
            compute_gidx(chunk, 0)
            fire_gather(0)
            pltpu.sync_copy(dst2d.at[pl.ds(idx_rowbase(0), K_BATCH)],
                            dstb.at[pl.ds(0, K_BATCH)])

            def step2(j2, carry):
                s0 = 2 * j2
                @pl.when(j2 > 0)
                def _():
                    drain_scatter(1)
                issue_idx(s0 + 1, 1)
                drain_gather(0)
                fire_scatter(0)
                wait_idx(1)
                compute_gidx(chunk, 1)
                fire_gather(1)
                drain_scatter(0)

                @pl.when(j2 < AGG_STEPS // 2 - 1)
                def _():
                    issue_idx(s0 + 2, 0)
                drain_gather(1)
                fire_scatter(1)

                @pl.when(j2 < AGG_STEPS // 2 - 1)
                def _():
                    wait_idx(0)
                    compute_gidx(chunk, 0)
                    fire_gather(0)
                return carry

            lax.fori_loop(0, AGG_STEPS // 2, step2, 0)
            drain_scatter(1)
            plsc.subcore_barrier()
            pltpu.sync_copy(
                acc.at[pl.ds(sid * STRIPE, STRIPE)],
                out.at[chunk, pl.ds(sid * STRIPE, STRIPE)])
            plsc.subcore_barrier()

        return carry

    lax.fori_loop(0, (nch + 1) // 2, chunk_body, 0)


@functools.lru_cache(maxsize=None)
def _agg_kernel(nch):
    return pl.kernel(
        functools.partial(_agg_body, nch),
        out_type=jax.ShapeDtypeStruct((nch, N_PAD, L), jnp.float32),
        mesh=_MESH,
        compiler_params=pltpu.CompilerParams(use_tc_tiling_on_sc=False),
        scratch_types=[
            pltpu.VMEM_SHARED((N_PAD, L), jnp.float32),
            pltpu.VMEM((2 * K_BATCH, 128), jnp.int32),
            pltpu.VMEM((2 * K_BATCH, 128), jnp.int32),
            pltpu.VMEM((2 * K_BATCH, 128), jnp.int32),
            pltpu.VMEM((2 * STEP_E, L), jnp.float32),
            pltpu.SemaphoreType.DMA,
            pltpu.SemaphoreType.DMA,
            pltpu.SemaphoreType.DMA,
        ],
    )


def _pool_body(emb, batch2d, zeros_pool, out, acc, rbuf, bidx):
    cid = lax.axis_index("c")
    sid = lax.axis_index("s")

    @pl.when(sid == 0)
    def _z():
        pltpu.sync_copy(zeros_pool, acc)

    plsc.subcore_barrier()
    wid = cid * NS + sid

    def step(j, carry):
        blk = wid + (NC * NS) * j

        @pl.when(blk < POOL_BLOCKS)
        def _():
            rowbase = blk * POOL_STEP
            pltpu.sync_copy(emb.at[pl.ds(rowbase, POOL_STEP)], rbuf)
            pltpu.sync_copy(batch2d.at[pl.ds(blk, 1)], bidx)
            pltpu.sync_copy(rbuf, acc.at[bidx.at[0]], add=True)

        return carry

    lax.fori_loop(0, POOL_STEPS, step, 0)
    plsc.subcore_barrier()
    rows_per_tile = POOL_ROWS // NS
    pltpu.sync_copy(acc.at[pl.ds(sid * rows_per_tile, rows_per_tile)],
                    out.at[cid, pl.ds(sid * rows_per_tile, rows_per_tile)])


def _pool_kernel(emb, batch2d, zeros_pool):
    return pl.kernel(
        _pool_body,
        out_type=jax.ShapeDtypeStruct((NC, POOL_ROWS, 128), jnp.float32),
        mesh=_MESH,
        compiler_params=pltpu.CompilerParams(use_tc_tiling_on_sc=False),
        scratch_types=[
            pltpu.VMEM_SHARED((POOL_ROWS, 128), jnp.float32),
            pltpu.VMEM((POOL_STEP, 128), jnp.float32),
            pltpu.VMEM((1, POOL_STEP), jnp.int32),
        ],
    )(emb, batch2d, zeros_pool)


def _prep_body(deg16_ref, x_ref, dinv_ref, p0_ref):
    i = pl.program_id(0)
    deg = deg16_ref[0, :, 0:1] + deg16_ref[1, :, 0:1] + 1.0
    rows = i * BN + lax.broadcasted_iota(jnp.int32, (BN, 1), 0)
    dinv = jnp.where(rows < N, lax.rsqrt(deg), 0.0)
    dinv_ref[...] = dinv
    p0_ref[...] = x_ref[...] * dinv


def _prep_kernel(deg16, x_pad):
    din = x_pad.shape[1]
    return pl.pallas_call(
        _prep_body,
        grid=(N_BLOCKS,),
        in_specs=[
            pl.BlockSpec((NC, BN, L), lambda i: (0, i, 0)),
            pl.BlockSpec((BN, din), lambda i: (i, 0)),
        ],
        out_specs=[
            pl.BlockSpec((BN, 1), lambda i: (i, 0)),
            pl.BlockSpec((BN, din), lambda i: (i, 0)),
        ],
        out_shape=[
            jax.ShapeDtypeStruct((N_PAD, 1), jnp.float32),
            jax.ShapeDtypeStruct((N_PAD, din), jnp.float32),
        ],
    )(deg16, x_pad)


def _layer_body(final, nch, u3_ref, p_ref, dinv_ref, w_ref, b_ref, gs_ref,
                be_ref, out_ref):
    c = pl.program_id(1)
    dinv = dinv_ref[...]

    @pl.when(c == 0)
    def _init():
        out_ref[...] = jnp.dot(p_ref[...] * dinv, w_ref[...],
                               preferred_element_type=jnp.float32) + b_ref[...]

    wc = w_ref[pl.ds(c * L, L), :]
    out_ref[...] += jnp.dot(u3_ref[0] * dinv, wc,
                            preferred_element_type=jnp.float32)

    @pl.when(c == nch - 1)
    def _fin():
        h = jnp.maximum(out_ref[...], 0.0) * gs_ref[...] + be_ref[...]
        out_ref[...] = h if final else h * dinv


def _layer_kernel(u3, p, dinv, w, b, gs, be, final):
    nch = u3.shape[0]
    din = p.shape[1]
    dout = w.shape[1]
    return pl.pallas_call(
        functools.partial(_layer_body, final, nch),
        grid=(N_BLOCKS, nch),
        in_specs=[
            pl.BlockSpec((1, BN, L), lambda i, c: (c, i, 0)),
            pl.BlockSpec((BN, din), lambda i, c: (i, 0)),
            pl.BlockSpec((BN, 1), lambda i, c: (i, 0)),
            pl.BlockSpec((din, dout), lambda i, c: (0, 0)),
            pl.BlockSpec((1, dout), lambda i, c: (0, 0)),
            pl.BlockSpec((1, dout), lambda i, c: (0, 0)),
            pl.BlockSpec((1, dout), lambda i, c: (0, 0)),
        ],
        out_specs=pl.BlockSpec((BN, dout), lambda i, c: (i, 0)),
        out_shape=jax.ShapeDtypeStruct((N_PAD, dout), jnp.float32),
    )(u3, p, dinv, w, b, gs, be)


def _emb_body(h_ref, w_ref, b_ref, out_ref):
    z = jnp.dot(h_ref[...], w_ref[...],
                preferred_element_type=jnp.float32) + b_ref[...]
    m = jnp.max(z, axis=-1, keepdims=True)
    e = jnp.exp(z - m)
    out_ref[...] = e / jnp.sum(e, axis=-1, keepdims=True)


def _emb_kernel(h5, w, b):
    din = h5.shape[1]
    return pl.pallas_call(
        _emb_body,
        grid=(N_BLOCKS,),
        in_specs=[
            pl.BlockSpec((BN, din), lambda i: (i, 0)),
            pl.BlockSpec((din, 128), lambda i: (0, 0)),
            pl.BlockSpec((1, 128), lambda i: (0, 0)),
        ],
        out_specs=pl.BlockSpec((BN, 128), lambda i: (i, 0)),
        out_shape=jax.ShapeDtypeStruct((N_PAD, 128), jnp.float32),
    )(h5, w, b)


def _head_body(pool_ref, wm0, bm0, gsm0, bem0, wm1, bm1, gsm1, bem1, wo, bo,
               out_ref):
    hg = pool_ref[0, :NUM_GRAPHS, :] + pool_ref[1, :NUM_GRAPHS, :]
    z1 = jnp.dot(hg, wm0[...], preferred_element_type=jnp.float32) + bm0[...]
    h1 = jnp.maximum(z1, 0.0) * gsm0[...] + bem0[...]
    z2 = jnp.dot(h1, wm1[...], preferred_element_type=jnp.float32) + bm1[...]
    h2 = jnp.maximum(z2, 0.0) * gsm1[...] + bem1[...]
    out_ref[...] = jnp.dot(h2, wo[...],
                           preferred_element_type=jnp.float32) + bo[...]


def _head_kernel(pool, wm0, bm0, gsm0, bem0, wm1, bm1, gsm1, bem1, wo, bo):
    full = lambda a: pl.BlockSpec(a.shape, lambda: tuple(0 for _ in a.shape))
    return pl.pallas_call(
        _head_body,
        in_specs=[full(pool), full(wm0), full(bm0), full(gsm0), full(bem0),
                  full(wm1), full(bm1), full(gsm1), full(bem1), full(wo),
                  full(bo)],
        out_specs=pl.BlockSpec((NUM_GRAPHS, 128), lambda: (0, 0)),
        out_shape=jax.ShapeDtypeStruct((NUM_GRAPHS, 128), jnp.float32),
    )(pool, wm0, bm0, gsm0, bem0, wm1, bm1, gsm1, bem1, wo, bo)


_BN_SCALE = 1.0 / (1.0 + 1e-5) ** 0.5
_CONV_PADS = [(48, 64), (64, 80), (80, 112), (112, 128), (128, 160)]


def kernel(x, edge_index, batch, W0, b0, g0, be0, W1, b1, g1, be1, W2, b2, g2,
           be2, W3, b3, g3, be3, W4, b4, g4, be4, Wemb, bemb, Wm0, bm0, gm0,
           bem0, Wm1, bm1, gm1, bem1, Wout, bout):
    f32 = jnp.float32
    x_pad = _pad2(x, N_PAD, 48)
    pad_e = E_PAD - E
    padidx = (N + (jnp.arange(pad_e, dtype=jnp.int32) % 16)).astype(jnp.int32)
    src2d = jnp.concatenate([edge_index[0], padidx]).reshape(E_PAD // 128, 128)
    dst2d = jnp.concatenate([edge_index[1], padidx]).reshape(E_PAD // 128, 128)
    batch2d = jnp.pad(batch, (0, N_PAD - N),
                      constant_values=NUM_GRAPHS).reshape(POOL_BLOCKS,
                                                          POOL_STEP)
    zeros16 = jnp.zeros((N_PAD, L), f32)
    zeros_pool = jnp.zeros((POOL_ROWS, 128), f32)

    convs = [(W0, b0, g0, be0), (W1, b1, g1, be1), (W2, b2, g2, be2),
             (W3, b3, g3, be3), (W4, b4, g4, be4)]

    deg16 = _deg_kernel(dst2d, zeros16)
    dinv, p = _prep_kernel(deg16, x_pad)

    for li, ((din_p, dout_p), (W, b, g, be)) in enumerate(zip(_CONV_PADS,
                                                              convs)):
        nch = din_p // L
        u = _agg_kernel(nch)(src2d, dst2d, p.reshape(N_PAD * nch, L),
                             zeros16)
        p = _layer_kernel(
            u, p, dinv,
            _pad2(W, din_p, dout_p), _row(b, dout_p),
            _row(g * _BN_SCALE, dout_p), _row(be, dout_p),
            final=(li == len(convs) - 1))

    wemb = _pad2(Wemb, 160, 128)
    bemb_p = jnp.full((1, 128), -1e30, f32).at[0, :100].set(bemb)
    emb = _emb_kernel(p, wemb, bemb_p)

    pool = _pool_kernel(emb, batch2d, zeros_pool)

    out_pad = _head_kernel(
        pool,
        _pad2(Wm0, 128, 64), _row(bm0, 64), _row(gm0 * _BN_SCALE, 64),
        _row(bem0, 64),
        _pad2(Wm1, 64, 32), _row(bm1, 32), _row(gm1 * _BN_SCALE, 32),
        _row(bem1, 32),
        _pad2(Wout, 32, 128), _row(bout, 128))
    return out_pad[:, :4]


# speedup vs baseline: 11.4346x; 1.1532x over previous
import functools

import jax
import jax.numpy as jnp
from jax import lax
from jax.experimental import pallas as pl
from jax.experimental.pallas import tpu as pltpu
from jax.experimental.pallas import tpu_sc as plsc

N = 100000
E = 1600000
NUM_GRAPHS = 512

NC = 2
NS = 16
L = 16

N_PAD = 100352
STRIPE = N_PAD // NS
BN = 2048
N_BLOCKS = N_PAD // BN

K_BATCH = 4
STEP_E = K_BATCH * 128
E_TOT = E + N
E_PAD = 1703936
ROWS_PER_TILE = (E_PAD // NS) // 128
AGG_STEPS = ROWS_PER_TILE // K_BATCH
DEG_STEPS = (E_PAD // (NC * NS)) // STEP_E

POOL_ROWS = 640
POOL_STEP = 128
POOL_BLOCKS = N_PAD // POOL_STEP
POOL_STEPS = -(-POOL_BLOCKS // (NC * NS))

_MESH = plsc.VectorSubcoreMesh(
    core_axis_name="c", subcore_axis_name="s", num_cores=NC, num_subcores=NS)


def _pad2(a, r, c):
    return jnp.pad(a, ((0, r - a.shape[0]), (0, c - a.shape[1])))


def _row(a, c):
    return jnp.pad(a, (0, c - a.shape[0])).reshape(1, c)


def _kron8(w):
    k, d = w.shape
    out = jnp.zeros((8, k, 8, d), w.dtype)
    out = out.at[jnp.arange(8), :, jnp.arange(8), :].set(
        jnp.broadcast_to(w, (8, k, d)))
    return out.reshape(8 * k, 8 * d)


def _deg_body(dst2d, zeros16, out, acc, dstb, ones_buf, isem, ssem):
    cid = lax.axis_index("c")
    sid = lax.axis_index("s")
    one_row = jnp.full((L,), 1.0, jnp.float32)
    for r in range(128):
        ones_buf[r, :] = one_row
    pltpu.sync_copy(zeros16.at[pl.ds(sid * STRIPE, STRIPE)],
                    acc.at[pl.ds(sid * STRIPE, STRIPE)])
    plsc.subcore_barrier()

    base_rows = (cid * NS + sid) * (DEG_STEPS * K_BATCH)

    def issue_idx(step, slot):
        pltpu.async_copy(dst2d.at[pl.ds(base_rows + step * K_BATCH, K_BATCH)],
                         dstb.at[pl.ds(slot * K_BATCH, K_BATCH)], isem)

    def wait_idx(slot):
        pltpu.make_async_copy(dst2d.at[pl.ds(0, K_BATCH)],
                              dstb.at[pl.ds(slot * K_BATCH, K_BATCH)],
                              isem).wait()

    def fire_scatter(slot):
        for b in range(K_BATCH):
            pltpu.async_copy(ones_buf, acc.at[dstb.at[slot * K_BATCH + b]],
                             ssem, add=True)

    def drain_scatter():
        for b in range(K_BATCH):
            pltpu.make_async_copy(ones_buf, acc.at[pl.ds(0, 128)],
                                  ssem).wait()

    issue_idx(0, 0)

    def step2(j2, carry):
        s0 = 2 * j2

        @pl.when(j2 > 0)
        def _():
            drain_scatter()
        issue_idx(s0 + 1, 1)
        wait_idx(0)
        fire_scatter(0)
        drain_scatter()

        @pl.when(j2 < DEG_STEPS // 2 - 1)
        def _():
            issue_idx(s0 + 2, 0)
        wait_idx(1)
        fire_scatter(1)
        return carry

    lax.fori_loop(0, DEG_STEPS // 2, step2, 0)
    drain_scatter()
    plsc.subcore_barrier()
    pltpu.sync_copy(acc.at[pl.ds(sid * STRIPE, STRIPE)],
                    out.at[cid, pl.ds(sid * STRIPE, STRIPE)])


def _deg_kernel(dst2d, zeros16):
    return pl.kernel(
        _deg_body,
        out_type=jax.ShapeDtypeStruct((NC, N_PAD, L), jnp.float32),
        mesh=_MESH,
        compiler_params=pltpu.CompilerParams(use_tc_tiling_on_sc=False),
        scratch_types=[
            pltpu.VMEM_SHARED((N_PAD, L), jnp.float32),
            pltpu.VMEM((2 * K_BATCH, 128), jnp.int32),
            pltpu.VMEM((128, L), jnp.float32),
            pltpu.SemaphoreType.DMA,
            pltpu.SemaphoreType.DMA,
        ],
    )(dst2d, zeros16)


def _agg_body(nch, src2d, dst2d, p2d, zeros16, out, acc, srcb, dstb, gidxb,
              rows, isem, gsem, ssem):
    cid = lax.axis_index("c")
    sid = lax.axis_index("s")

    def idx_rowbase(step):
        return sid * ROWS_PER_TILE + step * K_BATCH

    def issue_idx(step, slot):
        base = idx_rowbase(step)
        pltpu.async_copy(src2d.at[pl.ds(base, K_BATCH)],
                         srcb.at[pl.ds(slot * K_BATCH, K_BATCH)], isem)
        pltpu.async_copy(dst2d.at[pl.ds(base, K_BATCH)],
                         dstb.at[pl.ds(slot * K_BATCH, K_BATCH)], isem)

    def wait_idx(slot):
        for ref in (srcb, dstb):
            pltpu.make_async_copy(
                src2d.at[pl.ds(0, K_BATCH)],
                ref.at[pl.ds(slot * K_BATCH, K_BATCH)], isem).wait()

    def compute_gidx(chunk, slot):
        for v in range(K_BATCH):
            r = slot * K_BATCH + v
            for u in range(128 // L):
                s16 = srcb[r, pl.ds(u * L, L)]
                gidxb[r, pl.ds(u * L, L)] = s16 * nch + chunk

    def fire_gather(slot):
        for b in range(K_BATCH):
            pltpu.async_copy(
                p2d.at[gidxb.at[slot * K_BATCH + b]],
                rows.at[pl.ds((slot * K_BATCH + b) * 128, 128)], gsem)

    def drain_gather(slot):
        for b in range(K_BATCH):
            pltpu.make_async_copy(
                p2d.at[pl.ds(0, 128)],
                rows.at[pl.ds((slot * K_BATCH + b) * 128, 128)], gsem).wait()

    def fire_scatter(slot):
        for b in range(K_BATCH):
            pltpu.async_copy(
                rows.at[pl.ds((slot * K_BATCH + b) * 128, 128)],
                acc.at[dstb.at[slot * K_BATCH + b]], ssem, add=True)

    def drain_scatter(slot):
        for b in range(K_BATCH):
            pltpu.make_async_copy(
                rows.at[pl.ds((slot * K_BATCH + b) * 128, 128)],
                acc.at[pl.ds(0, 128)], ssem).wait()

    def chunk_body(ci, carry):
        chunk = cid + 2 * ci

        @pl.when(chunk < nch)
        def _chunk():
            pltpu.sync_copy(zeros16.at[pl.ds(sid * STRIPE, STRIPE)],
                            acc.at[pl.ds(sid * STRIPE, STRIPE)])
            plsc.subcore_barrier()

            pltpu.sync_copy(src2d.at[pl.ds(idx_rowbase(0), K_BATCH)],
                            srcb.at[pl.ds(0, K_BATCH)])
            compute_gidx(chunk, 0)
            fire_gather(0)
            pltpu.sync_copy(dst2d.at[pl.ds(idx_rowbase(0), K_BATCH)],
                            dstb.at[pl.ds(0, K_BATCH)])

            def step2(j2, carry2):
                s0 = 2 * j2
                @pl.when(j2 > 0)
                def _():
                    drain_scatter(1)
                issue_idx(s0 + 1, 1)
                drain_gather(0)
                fire_scatter(0)
                wait_idx(1)
                compute_gidx(chunk, 1)
                fire_gather(1)
                drain_scatter(0)

                @pl.when(j2 < AGG_STEPS // 2 - 1)
                def _():
                    issue_idx(s0 + 2, 0)
                drain_gather(1)
                fire_scatter(1)

                @pl.when(j2 < AGG_STEPS // 2 - 1)
                def _():
                    wait_idx(0)
                    compute_gidx(chunk, 0)
                    fire_gather(0)
                return carry2

            lax.fori_loop(0, AGG_STEPS // 2, step2, 0)
            drain_scatter(1)
            plsc.subcore_barrier()
            pltpu.sync_copy(
                acc.at[pl.ds(sid * STRIPE, STRIPE)],
                out.at[chunk, pl.ds(sid * STRIPE, STRIPE)])
            plsc.subcore_barrier()

        return carry

    lax.fori_loop(0, (nch + 1) // 2, chunk_body, 0)


@functools.lru_cache(maxsize=None)
def _agg_kernel(nch):
    return pl.kernel(
        functools.partial(_agg_body, nch),
        out_type=jax.ShapeDtypeStruct((nch, N_PAD, L), jnp.float32),
        mesh=_MESH,
        compiler_params=pltpu.CompilerParams(use_tc_tiling_on_sc=False),
        scratch_types=[
            pltpu.VMEM_SHARED((N_PAD, L), jnp.float32),
            pltpu.VMEM((2 * K_BATCH, 128), jnp.int32),
            pltpu.VMEM((2 * K_BATCH, 128), jnp.int32),
            pltpu.VMEM((2 * K_BATCH, 128), jnp.int32),
            pltpu.VMEM((2 * STEP_E, L), jnp.float32),
            pltpu.SemaphoreType.DMA,
            pltpu.SemaphoreType.DMA,
            pltpu.SemaphoreType.DMA,
        ],
    )


def _pool_body(emb, batch2d, zeros_pool, out, acc, rbuf, bidx):
    cid = lax.axis_index("c")
    sid = lax.axis_index("s")

    @pl.when(sid == 0)
    def _z():
        pltpu.sync_copy(zeros_pool, acc)

    plsc.subcore_barrier()
    wid = cid * NS + sid

    def step(j, carry):
        blk = wid + (NC * NS) * j

        @pl.when(blk < POOL_BLOCKS)
        def _():
            rowbase = blk * POOL_STEP
            pltpu.sync_copy(emb.at[pl.ds(rowbase, POOL_STEP)], rbuf)
            pltpu.sync_copy(batch2d.at[pl.ds(blk, 1)], bidx)
            pltpu.sync_copy(rbuf, acc.at[bidx.at[0]], add=True)

        return carry

    lax.fori_loop(0, POOL_STEPS, step, 0)
    plsc.subcore_barrier()
    rows_per_tile = POOL_ROWS // NS
    pltpu.sync_copy(acc.at[pl.ds(sid * rows_per_tile, rows_per_tile)],
                    out.at[cid, pl.ds(sid * rows_per_tile, rows_per_tile)])


def _pool_kernel(emb, batch2d, zeros_pool):
    return pl.kernel(
        _pool_body,
        out_type=jax.ShapeDtypeStruct((NC, POOL_ROWS, 128), jnp.float32),
        mesh=_MESH,
        compiler_params=pltpu.CompilerParams(use_tc_tiling_on_sc=False),
        scratch_types=[
            pltpu.VMEM_SHARED((POOL_ROWS, 128), jnp.float32),
            pltpu.VMEM((POOL_STEP, 128), jnp.float32),
            pltpu.VMEM((1, POOL_STEP), jnp.int32),
        ],
    )(emb, batch2d, zeros_pool)


def _prep8_body(deg8_ref, dinv8_ref):
    deg = deg8_ref[0] + deg8_ref[1]
    dinv8_ref[...] = jnp.where(deg > 0, lax.rsqrt(deg), 0.0)


def _prep8_kernel(deg8):
    return pl.pallas_call(
        _prep8_body,
        grid=(N_BLOCKS,),
        in_specs=[pl.BlockSpec((NC, BN // 8, 128), lambda i: (0, i, 0))],
        out_specs=pl.BlockSpec((BN // 8, 128), lambda i: (i, 0)),
        out_shape=jax.ShapeDtypeStruct((N_PAD // 8, 128), jnp.float32),
    )(deg8)


def _prep0_body(deg16_ref, x_ref, p0_ref):
    deg = deg16_ref[0, :, 0:1] + deg16_ref[1, :, 0:1]
    dinv = jnp.where(deg > 0, lax.rsqrt(deg), 0.0)
    p0_ref[...] = x_ref[...] * dinv


def _prep0_kernel(deg16, x_pad):
    din = x_pad.shape[1]
    return pl.pallas_call(
        _prep0_body,
        grid=(N_BLOCKS,),
        in_specs=[
            pl.BlockSpec((NC, BN, L), lambda i: (0, i, 0)),
            pl.BlockSpec((BN, din), lambda i: (i, 0)),
        ],
        out_specs=pl.BlockSpec((BN, din), lambda i: (i, 0)),
        out_shape=jax.ShapeDtypeStruct((N_PAD, din), jnp.float32),
    )(deg16, x_pad)


def _layer_body(final, nch, u8_ref, dinv8_ref, w8_ref, bd_ref, b8_ref,
                gs8_ref, be8_ref, out_ref, zs, dv):
    c = pl.program_id(1)

    @pl.when(c == 0)
    def _init():
        zs[...] = jnp.broadcast_to(b8_ref[...], zs.shape)
        if not final:
            dv[...] = jnp.dot(dinv8_ref[...], bd_ref[...],
                              preferred_element_type=jnp.float32)

    zs[...] += jnp.dot(u8_ref[0] * dinv8_ref[...], w8_ref[0],
                       preferred_element_type=jnp.float32)

    @pl.when(c == nch - 1)
    def _fin():
        h = jnp.maximum(zs[...], 0.0) * gs8_ref[...] + be8_ref[...]
        if not final:
            h = h * dv[...]
        out_ref[...] = h


def _layer_kernel(u8, dinv8, w8, bd, b8, gs8, be8, final):
    nch = u8.shape[0]
    dout8 = w8.shape[2]
    return pl.pallas_call(
        functools.partial(_layer_body, final, nch),
        grid=(N_BLOCKS, nch),
        in_specs=[
            pl.BlockSpec((1, BN // 8, 128), lambda i, c: (c, i, 0)),
            pl.BlockSpec((BN // 8, 128), lambda i, c: (i, 0)),
            pl.BlockSpec((1, 128, dout8), lambda i, c: (c, 0, 0)),
            pl.BlockSpec((128, dout8), lambda i, c: (0, 0)),
            pl.BlockSpec((1, dout8), lambda i, c: (0, 0)),
            pl.BlockSpec((1, dout8), lambda i, c: (0, 0)),
            pl.BlockSpec((1, dout8), lambda i, c: (0, 0)),
        ],
        out_specs=pl.BlockSpec((BN // 8, dout8), lambda i, c: (i, 0)),
        out_shape=jax.ShapeDtypeStruct((N_PAD // 8, dout8), jnp.float32),
        scratch_shapes=[pltpu.VMEM((BN // 8, dout8), jnp.float32),
                        pltpu.VMEM((BN // 8, dout8), jnp.float32)],
    )(u8, dinv8, w8, bd, b8, gs8, be8)


def _emb_body(h8_ref, w8_ref, b8_ref, out_ref):
    z8 = jnp.dot(h8_ref[...], w8_ref[...],
                 preferred_element_type=jnp.float32) + b8_ref[...]
    for k in range(8):
        zk = z8[:, k * 128:(k + 1) * 128]
        m = jnp.max(zk, axis=-1, keepdims=True)
        e = jnp.exp(zk - m)
        out_ref[:, k * 128:(k + 1) * 128] = e / jnp.sum(e, axis=-1,
                                                        keepdims=True)


def _emb_kernel(h8, w8, b8):
    din8 = h8.shape[1]
    return pl.pallas_call(
        _emb_body,
        grid=(N_BLOCKS,),
        in_specs=[
            pl.BlockSpec((BN // 8, din8), lambda i: (i, 0)),
            pl.BlockSpec((din8, 1024), lambda i: (0, 0)),
            pl.BlockSpec((1, 1024), lambda i: (0, 0)),
        ],
        out_specs=pl.BlockSpec((BN // 8, 1024), lambda i: (i, 0)),
        out_shape=jax.ShapeDtypeStruct((N_PAD // 8, 1024), jnp.float32),
    )(h8, w8, b8)


def _head_body(pool_ref, wm0, bm0, gsm0, bem0, wm1, bm1, gsm1, bem1, wo, bo,
               out_ref):
    hg = pool_ref[0, :NUM_GRAPHS, :] + pool_ref[1, :NUM_GRAPHS, :]
    z1 = jnp.dot(hg, wm0[...], preferred_element_type=jnp.float32) + bm0[...]
    h1 = jnp.maximum(z1, 0.0) * gsm0[...] + bem0[...]
    z2 = jnp.dot(h1, wm1[...], preferred_element_type=jnp.float32) + bm1[...]
    h2 = jnp.maximum(z2, 0.0) * gsm1[...] + bem1[...]
    out_ref[...] = jnp.dot(h2, wo[...],
                           preferred_element_type=jnp.float32) + bo[...]


def _head_kernel(pool, wm0, bm0, gsm0, bem0, wm1, bm1, gsm1, bem1, wo, bo):
    full = lambda a: pl.BlockSpec(a.shape, lambda: tuple(0 for _ in a.shape))
    return pl.pallas_call(
        _head_body,
        in_specs=[full(pool), full(wm0), full(bm0), full(gsm0), full(bem0),
                  full(wm1), full(bm1), full(gsm1), full(bem1), full(wo),
                  full(bo)],
        out_specs=pl.BlockSpec((NUM_GRAPHS, 128), lambda: (0, 0)),
        out_shape=jax.ShapeDtypeStruct((NUM_GRAPHS, 128), jnp.float32),
    )(pool, wm0, bm0, gsm0, bem0, wm1, bm1, gsm1, bem1, wo, bo)


_BN_SCALE = 1.0 / (1.0 + 1e-5) ** 0.5
_CONV_PADS = [(48, 64), (64, 80), (80, 112), (112, 128), (128, 160)]


def kernel(x, edge_index, batch, W0, b0, g0, be0, W1, b1, g1, be1, W2, b2, g2,
           be2, W3, b3, g3, be3, W4, b4, g4, be4, Wemb, bemb, Wm0, bm0, gm0,
           bem0, Wm1, bm1, gm1, bem1, Wout, bout):
    f32 = jnp.float32
    x_pad = _pad2(x, N_PAD, 48)
    loop = jnp.arange(N, dtype=jnp.int32)
    padidx = (N + (jnp.arange(E_PAD - E_TOT, dtype=jnp.int32) % 16))
    src2d = jnp.concatenate([edge_index[0], loop, padidx]).reshape(-1, 128)
    dst2d = jnp.concatenate([edge_index[1], loop, padidx]).reshape(-1, 128)
    batch2d = jnp.pad(batch, (0, N_PAD - N),
                      constant_values=NUM_GRAPHS).reshape(POOL_BLOCKS,
                                                          POOL_STEP)
    zeros16 = jnp.zeros((N_PAD, L), f32)
    zeros_pool = jnp.zeros((POOL_ROWS, 128), f32)

    convs = [(W0, b0, g0, be0), (W1, b1, g1, be1), (W2, b2, g2, be2),
             (W3, b3, g3, be3), (W4, b4, g4, be4)]

    deg16 = _deg_kernel(dst2d, zeros16)
    dinv8 = _prep8_kernel(deg16.reshape(NC, N_PAD // 8, 128))
    p = _prep0_kernel(deg16, x_pad)
    p2d = p.reshape(N_PAD * 3, L)

    for li, ((din_p, dout_p), (W, b, g, be)) in enumerate(zip(_CONV_PADS,
                                                              convs)):
        nch = din_p // L
        u = _agg_kernel(nch)(src2d, dst2d, p2d, zeros16)
        u8 = u.reshape(nch, N_PAD // 8, 128)
        wp = _pad2(W, din_p, dout_p)
        w8 = jnp.stack([_kron8(wp[c * L:(c + 1) * L, :]) for c in range(nch)])
        p8 = _layer_kernel(
            u8, dinv8, w8,
            _kron8(jnp.full((L, dout_p), 1.0 / L, f32)),
            jnp.tile(_row(b, dout_p), (1, 8)),
            jnp.tile(_row(g * _BN_SCALE, dout_p), (1, 8)),
            jnp.tile(_row(be, dout_p), (1, 8)),
            final=(li == len(convs) - 1))
        p2d = p8.reshape(N_PAD * (dout_p // L), L)

    wemb8 = _kron8(_pad2(Wemb, 160, 128))
    bemb_p = jnp.full((128,), -1e30, f32).at[:100].set(bemb)
    emb8 = _emb_kernel(p8, wemb8, jnp.tile(bemb_p, 8).reshape(1, 1024))

    pool = _pool_kernel(emb8.reshape(N_PAD, 128), batch2d, zeros_pool)

    out_pad = _head_kernel(
        pool,
        _pad2(Wm0, 128, 64), _row(bm0, 64), _row(gm0 * _BN_SCALE, 64),
        _row(bem0, 64),
        _pad2(Wm1, 64, 32), _row(bm1, 32), _row(gm1 * _BN_SCALE, 32),
        _row(bem1, 32),
        _pad2(Wout, 32, 128), _row(bout, 128))
    return out_pad[:, :4]


# speedup vs baseline: 11.8828x; 1.0392x over previous
import functools

import jax
import jax.numpy as jnp
from jax import lax
from jax.experimental import pallas as pl
from jax.experimental.pallas import tpu as pltpu
from jax.experimental.pallas import tpu_sc as plsc

N = 100000
E = 1600000
NUM_GRAPHS = 512

NC = 2
NS = 16
L = 16

N_PAD = 100352
STRIPE = N_PAD // NS
BN = 2048
N_BLOCKS = N_PAD // BN

K_BATCH = 4
STEP_E = K_BATCH * 128
E_TOT = E + N
E_PAD = 1703936
ROWS_PER_TILE = (E_PAD // NS) // 128
AGG_STEPS = ROWS_PER_TILE // K_BATCH
DEG_STEPS = (E_PAD // (NC * NS)) // STEP_E

POOL_ROWS = 640
POOL_STEP = 128
POOL_BLOCKS = N_PAD // POOL_STEP
POOL_STEPS = -(-POOL_BLOCKS // (NC * NS))

_MESH = plsc.VectorSubcoreMesh(
    core_axis_name="c", subcore_axis_name="s", num_cores=NC, num_subcores=NS)


def _pad2(a, r, c):
    return jnp.pad(a, ((0, r - a.shape[0]), (0, c - a.shape[1])))


def _row(a, c):
    return jnp.pad(a, (0, c - a.shape[0])).reshape(1, c)


def _kron8(w):
    k, d = w.shape
    out = jnp.zeros((8, k, 8, d), w.dtype)
    out = out.at[jnp.arange(8), :, jnp.arange(8), :].set(
        jnp.broadcast_to(w, (8, k, d)))
    return out.reshape(8 * k, 8 * d)


def _deg_body(dst2d, zeros16, out, acc, dstb, ones_buf, isem, ssem):
    cid = lax.axis_index("c")
    sid = lax.axis_index("s")
    one_row = jnp.full((L,), 1.0, jnp.float32)
    for r in range(128):
        ones_buf[r, :] = one_row
    pltpu.sync_copy(zeros16.at[pl.ds(sid * STRIPE, STRIPE)],
                    acc.at[pl.ds(sid * STRIPE, STRIPE)])
    plsc.subcore_barrier()

    base_rows = (cid * NS + sid) * (DEG_STEPS * K_BATCH)

    def issue_idx(step, slot):
        pltpu.async_copy(dst2d.at[pl.ds(base_rows + step * K_BATCH, K_BATCH)],
                         dstb.at[pl.ds(slot * K_BATCH, K_BATCH)], isem)

    def wait_idx(slot):
        pltpu.make_async_copy(dst2d.at[pl.ds(0, K_BATCH)],
                              dstb.at[pl.ds(slot * K_BATCH, K_BATCH)],
                              isem).wait()

    def fire_scatter(slot):
        for b in range(K_BATCH):
            pltpu.async_copy(ones_buf, acc.at[dstb.at[slot * K_BATCH + b]],
                             ssem, add=True)

    def drain_scatter():
        for b in range(K_BATCH):
            pltpu.make_async_copy(ones_buf, acc.at[pl.ds(0, 128)],
                                  ssem).wait()

    issue_idx(0, 0)

    def step2(j2, carry):
        s0 = 2 * j2

        @pl.when(j2 > 0)
        def _():
            drain_scatter()
        issue_idx(s0 + 1, 1)
        wait_idx(0)
        fire_scatter(0)
        drain_scatter()

        @pl.when(j2 < DEG_STEPS // 2 - 1)
        def _():
            issue_idx(s0 + 2, 0)
        wait_idx(1)
        fire_scatter(1)
        return carry

    lax.fori_loop(0, DEG_STEPS // 2, step2, 0)
    drain_scatter()
    plsc.subcore_barrier()
    pltpu.sync_copy(acc.at[pl.ds(sid * STRIPE, STRIPE)],
                    out.at[cid, pl.ds(sid * STRIPE, STRIPE)])


def _deg_kernel(dst2d, zeros16):
    return pl.kernel(
        _deg_body,
        out_type=jax.ShapeDtypeStruct((NC, N_PAD, L), jnp.float32),
        mesh=_MESH,
        compiler_params=pltpu.CompilerParams(use_tc_tiling_on_sc=False),
        scratch_types=[
            pltpu.VMEM_SHARED((N_PAD, L), jnp.float32),
            pltpu.VMEM((2 * K_BATCH, 128), jnp.int32),
            pltpu.VMEM((128, L), jnp.float32),
            pltpu.SemaphoreType.DMA,
            pltpu.SemaphoreType.DMA,
        ],
    )(dst2d, zeros16)


def _agg_body(nch, src2d, dst2d, p2d, zeros16, out, acc, srcb, dstb, gidxb,
              rows, isem, gsem, ssem):
    cid = lax.axis_index("c")
    sid = lax.axis_index("s")

    def idx_rowbase(step):
        return sid * ROWS_PER_TILE + step * K_BATCH

    def issue_idx(step, slot):
        base = idx_rowbase(step)
        pltpu.async_copy(src2d.at[pl.ds(base, K_BATCH)],
                         srcb.at[pl.ds(slot * K_BATCH, K_BATCH)], isem)
        pltpu.async_copy(dst2d.at[pl.ds(base, K_BATCH)],
                         dstb.at[pl.ds(slot * K_BATCH, K_BATCH)], isem)

    def wait_idx(slot):
        for ref in (srcb, dstb):
            pltpu.make_async_copy(
                src2d.at[pl.ds(0, K_BATCH)],
                ref.at[pl.ds(slot * K_BATCH, K_BATCH)], isem).wait()

    def compute_gidx(chunk, slot):
        for v in range(K_BATCH):
            r = slot * K_BATCH + v
            for u in range(128 // L):
                s16 = srcb[r, pl.ds(u * L, L)]
                gidxb[r, pl.ds(u * L, L)] = s16 * nch + chunk

    def fire_gather(slot):
        for b in range(K_BATCH):
            pltpu.async_copy(
                p2d.at[gidxb.at[slot * K_BATCH + b]],
                rows.at[pl.ds((slot * K_BATCH + b) * 128, 128)], gsem)

    def drain_gather(slot):
        for b in range(K_BATCH):
            pltpu.make_async_copy(
                p2d.at[pl.ds(0, 128)],
                rows.at[pl.ds((slot * K_BATCH + b) * 128, 128)], gsem).wait()

    def fire_scatter(slot):
        for b in range(K_BATCH):
            pltpu.async_copy(
                rows.at[pl.ds((slot * K_BATCH + b) * 128, 128)],
                acc.at[dstb.at[slot * K_BATCH + b]], ssem, add=True)

    def drain_scatter(slot):
        for b in range(K_BATCH):
            pltpu.make_async_copy(
                rows.at[pl.ds((slot * K_BATCH + b) * 128, 128)],
                acc.at[pl.ds(0, 128)], ssem).wait()

    def chunk_body(ci, carry):
        chunk = cid + 2 * ci

        @pl.when(chunk < nch)
        def _chunk():
            pltpu.sync_copy(zeros16.at[pl.ds(sid * STRIPE, STRIPE)],
                            acc.at[pl.ds(sid * STRIPE, STRIPE)])
            plsc.subcore_barrier()

            pltpu.sync_copy(src2d.at[pl.ds(idx_rowbase(0), K_BATCH)],
                            srcb.at[pl.ds(0, K_BATCH)])
            compute_gidx(chunk, 0)
            fire_gather(0)
            pltpu.sync_copy(dst2d.at[pl.ds(idx_rowbase(0), K_BATCH)],
                            dstb.at[pl.ds(0, K_BATCH)])

            def step2(j2, carry2):
                s0 = 2 * j2
                @pl.when(j2 > 0)
                def _():
                    drain_scatter(1)
                issue_idx(s0 + 1, 1)
                drain_gather(0)
                fire_scatter(0)
                wait_idx(1)
                compute_gidx(chunk, 1)
                fire_gather(1)
                drain_scatter(0)

                @pl.when(j2 < AGG_STEPS // 2 - 1)
                def _():
                    issue_idx(s0 + 2, 0)
                drain_gather(1)
                fire_scatter(1)

                @pl.when(j2 < AGG_STEPS // 2 - 1)
                def _():
                    wait_idx(0)
                    compute_gidx(chunk, 0)
                    fire_gather(0)
                return carry2

            lax.fori_loop(0, AGG_STEPS // 2, step2, 0)
            drain_scatter(1)
            plsc.subcore_barrier()
            pltpu.sync_copy(
                acc.at[pl.ds(sid * STRIPE, STRIPE)],
                out.at[chunk, pl.ds(sid * STRIPE, STRIPE)])
            plsc.subcore_barrier()

        return carry

    lax.fori_loop(0, (nch + 1) // 2, chunk_body, 0)


@functools.lru_cache(maxsize=None)
def _agg_kernel(nch):
    return pl.kernel(
        functools.partial(_agg_body, nch),
        out_type=jax.ShapeDtypeStruct((nch, N_PAD, L), jnp.float32),
        mesh=_MESH,
        compiler_params=pltpu.CompilerParams(use_tc_tiling_on_sc=False),
        scratch_types=[
            pltpu.VMEM_SHARED((N_PAD, L), jnp.float32),
            pltpu.VMEM((2 * K_BATCH, 128), jnp.int32),
            pltpu.VMEM((2 * K_BATCH, 128), jnp.int32),
            pltpu.VMEM((2 * K_BATCH, 128), jnp.int32),
            pltpu.VMEM((2 * STEP_E, L), jnp.float32),
            pltpu.SemaphoreType.DMA,
            pltpu.SemaphoreType.DMA,
            pltpu.SemaphoreType.DMA,
        ],
    )


def _pool_body(emb, batch2d, zeros_pool, out, acc, rbuf, bidx):
    cid = lax.axis_index("c")
    sid = lax.axis_index("s")

    @pl.when(sid == 0)
    def _z():
        pltpu.sync_copy(zeros_pool, acc)

    plsc.subcore_barrier()
    wid = cid * NS + sid

    def step(j, carry):
        blk = wid + (NC * NS) * j

        @pl.when(blk < POOL_BLOCKS)
        def _():
            rowbase = blk * POOL_STEP
            pltpu.sync_copy(emb.at[pl.ds(rowbase, POOL_STEP)], rbuf)
            pltpu.sync_copy(batch2d.at[pl.ds(blk, 1)], bidx)
            pltpu.sync_copy(rbuf, acc.at[bidx.at[0]], add=True)

        return carry

    lax.fori_loop(0, POOL_STEPS, step, 0)
    plsc.subcore_barrier()
    rows_per_tile = POOL_ROWS // NS
    pltpu.sync_copy(acc.at[pl.ds(sid * rows_per_tile, rows_per_tile)],
                    out.at[cid, pl.ds(sid * rows_per_tile, rows_per_tile)])


def _pool_kernel(emb, batch2d, zeros_pool):
    return pl.kernel(
        _pool_body,
        out_type=jax.ShapeDtypeStruct((NC, POOL_ROWS, 128), jnp.float32),
        mesh=_MESH,
        compiler_params=pltpu.CompilerParams(use_tc_tiling_on_sc=False),
        scratch_types=[
            pltpu.VMEM_SHARED((POOL_ROWS, 128), jnp.float32),
            pltpu.VMEM((POOL_STEP, 128), jnp.float32),
            pltpu.VMEM((1, POOL_STEP), jnp.int32),
        ],
    )(emb, batch2d, zeros_pool)


def _prep_body(deg8_ref, x8_ref, b48_ref, dinv8_ref, p08_ref):
    deg = deg8_ref[0] + deg8_ref[1]
    dinv8 = jnp.where(deg > 0, lax.rsqrt(deg), 0.0)
    dinv8_ref[...] = dinv8
    dv48 = jnp.dot(dinv8, b48_ref[...], preferred_element_type=jnp.float32)
    p08_ref[...] = x8_ref[...] * dv48


def _prep_kernel(deg8, x8):
    return pl.pallas_call(
        _prep_body,
        grid=(N_BLOCKS,),
        in_specs=[
            pl.BlockSpec((NC, BN // 8, 128), lambda i: (0, i, 0)),
            pl.BlockSpec((BN // 8, 384), lambda i: (i, 0)),
            pl.BlockSpec((128, 384), lambda i: (0, 0)),
        ],
        out_specs=[
            pl.BlockSpec((BN // 8, 128), lambda i: (i, 0)),
            pl.BlockSpec((BN // 8, 384), lambda i: (i, 0)),
        ],
        out_shape=[
            jax.ShapeDtypeStruct((N_PAD // 8, 128), jnp.float32),
            jax.ShapeDtypeStruct((N_PAD // 8, 384), jnp.float32),
        ],
    )(deg8, x8, _kron8(jnp.full((L, 48), 1.0 / L, jnp.float32)))


def _layer_body(final, nch, u8_ref, dinv8_ref, w8_ref, bd_ref, b8_ref,
                gs8_ref, be8_ref, out_ref, zs, dv):
    c = pl.program_id(1)

    @pl.when(c == 0)
    def _init():
        zs[...] = jnp.broadcast_to(b8_ref[...], zs.shape)
        if not final:
            dv[...] = jnp.dot(dinv8_ref[...], bd_ref[...],
                              preferred_element_type=jnp.float32)

    wc = w8_ref[pl.ds(c, 1)][0]
    zs[...] += jnp.dot(u8_ref[0] * dinv8_ref[...], wc,
                       preferred_element_type=jnp.float32)

    @pl.when(c == nch - 1)
    def _fin():
        h = jnp.maximum(zs[...], 0.0) * gs8_ref[...] + be8_ref[...]
        if not final:
            h = h * dv[...]
        out_ref[...] = h


def _layer_kernel(u8, dinv8, w8, bd, b8, gs8, be8, final):
    nch = u8.shape[0]
    dout8 = w8.shape[2]
    return pl.pallas_call(
        functools.partial(_layer_body, final, nch),
        grid=(N_BLOCKS, nch),
        in_specs=[
            pl.BlockSpec((1, BN // 8, 128), lambda i, c: (c, i, 0)),
            pl.BlockSpec((BN // 8, 128), lambda i, c: (i, 0)),
            pl.BlockSpec((nch, 128, dout8), lambda i, c: (0, 0, 0)),
            pl.BlockSpec((128, dout8), lambda i, c: (0, 0)),
            pl.BlockSpec((1, dout8), lambda i, c: (0, 0)),
            pl.BlockSpec((1, dout8), lambda i, c: (0, 0)),
            pl.BlockSpec((1, dout8), lambda i, c: (0, 0)),
        ],
        out_specs=pl.BlockSpec((BN // 8, dout8), lambda i, c: (i, 0)),
        out_shape=jax.ShapeDtypeStruct((N_PAD // 8, dout8), jnp.float32),
        scratch_shapes=[pltpu.VMEM((BN // 8, dout8), jnp.float32),
                        pltpu.VMEM((BN // 8, dout8), jnp.float32)],
    )(u8, dinv8, w8, bd, b8, gs8, be8)


def _emb_body(h8_ref, w8_ref, b8_ref, out_ref):
    z8 = jnp.dot(h8_ref[...], w8_ref[...],
                 preferred_element_type=jnp.float32) + b8_ref[...]
    for k in range(8):
        zk = z8[:, k * 128:(k + 1) * 128]
        m = jnp.max(zk, axis=-1, keepdims=True)
        e = jnp.exp(zk - m)
        out_ref[:, k * 128:(k + 1) * 128] = e / jnp.sum(e, axis=-1,
                                                        keepdims=True)


def _emb_kernel(h8, w8, b8):
    din8 = h8.shape[1]
    return pl.pallas_call(
        _emb_body,
        grid=(N_BLOCKS,),
        in_specs=[
            pl.BlockSpec((BN // 8, din8), lambda i: (i, 0)),
            pl.BlockSpec((din8, 1024), lambda i: (0, 0)),
            pl.BlockSpec((1, 1024), lambda i: (0, 0)),
        ],
        out_specs=pl.BlockSpec((BN // 8, 1024), lambda i: (i, 0)),
        out_shape=jax.ShapeDtypeStruct((N_PAD // 8, 1024), jnp.float32),
    )(h8, w8, b8)


def _head_body(pool_ref, wm0, bm0, gsm0, bem0, wm1, bm1, gsm1, bem1, wo, bo,
               out_ref):
    hg = pool_ref[0, :NUM_GRAPHS, :] + pool_ref[1, :NUM_GRAPHS, :]
    z1 = jnp.dot(hg, wm0[...], preferred_element_type=jnp.float32) + bm0[...]
    h1 = jnp.maximum(z1, 0.0) * gsm0[...] + bem0[...]
    z2 = jnp.dot(h1, wm1[...], preferred_element_type=jnp.float32) + bm1[...]
    h2 = jnp.maximum(z2, 0.0) * gsm1[...] + bem1[...]
    out_ref[...] = jnp.dot(h2, wo[...],
                           preferred_element_type=jnp.float32) + bo[...]


def _head_kernel(pool, wm0, bm0, gsm0, bem0, wm1, bm1, gsm1, bem1, wo, bo):
    full = lambda a: pl.BlockSpec(a.shape, lambda: tuple(0 for _ in a.shape))
    return pl.pallas_call(
        _head_body,
        in_specs=[full(pool), full(wm0), full(bm0), full(gsm0), full(bem0),
                  full(wm1), full(bm1), full(gsm1), full(bem1), full(wo),
                  full(bo)],
        out_specs=pl.BlockSpec((NUM_GRAPHS, 128), lambda: (0, 0)),
        out_shape=jax.ShapeDtypeStruct((NUM_GRAPHS, 128), jnp.float32),
    )(pool, wm0, bm0, gsm0, bem0, wm1, bm1, gsm1, bem1, wo, bo)


_BN_SCALE = 1.0 / (1.0 + 1e-5) ** 0.5
_CONV_PADS = [(48, 64), (64, 80), (80, 112), (112, 128), (128, 160)]


def kernel(x, edge_index, batch, W0, b0, g0, be0, W1, b1, g1, be1, W2, b2, g2,
           be2, W3, b3, g3, be3, W4, b4, g4, be4, Wemb, bemb, Wm0, bm0, gm0,
           bem0, Wm1, bm1, gm1, bem1, Wout, bout):
    f32 = jnp.float32
    x_pad = _pad2(x, N_PAD, 48)
    loop = jnp.arange(N, dtype=jnp.int32)
    padidx = (N + (jnp.arange(E_PAD - E_TOT, dtype=jnp.int32) % 16))
    src2d = jnp.concatenate([edge_index[0], loop, padidx]).reshape(-1, 128)
    dst2d = jnp.concatenate([edge_index[1], loop, padidx]).reshape(-1, 128)
    batch2d = jnp.pad(batch, (0, N_PAD - N),
                      constant_values=NUM_GRAPHS).reshape(POOL_BLOCKS,
                                                          POOL_STEP)
    zeros16 = jnp.zeros((N_PAD, L), f32)
    zeros_pool = jnp.zeros((POOL_ROWS, 128), f32)

    convs = [(W0, b0, g0, be0), (W1, b1, g1, be1), (W2, b2, g2, be2),
             (W3, b3, g3, be3), (W4, b4, g4, be4)]

    deg16 = _deg_kernel(dst2d, zeros16)
    dinv8, p08 = _prep_kernel(deg16.reshape(NC, N_PAD // 8, 128),
                              x_pad.reshape(N_PAD // 8, 384))
    p2d = p08.reshape(N_PAD * 3, L)

    for li, ((din_p, dout_p), (W, b, g, be)) in enumerate(zip(_CONV_PADS,
                                                              convs)):
        nch = din_p // L
        u = _agg_kernel(nch)(src2d, dst2d, p2d, zeros16)
        u8 = u.reshape(nch, N_PAD // 8, 128)
        wp = _pad2(W, din_p, dout_p)
        w8 = jnp.stack([_kron8(wp[c * L:(c + 1) * L, :]) for c in range(nch)])
        p8 = _layer_kernel(
            u8, dinv8, w8,
            _kron8(jnp.full((L, dout_p), 1.0 / L, f32)),
            jnp.tile(_row(b, dout_p), (1, 8)),
            jnp.tile(_row(g * _BN_SCALE, dout_p), (1, 8)),
            jnp.tile(_row(be, dout_p), (1, 8)),
            final=(li == len(convs) - 1))
        p2d = p8.reshape(N_PAD * (dout_p // L), L)

    wemb8 = _kron8(_pad2(Wemb, 160, 128))
    bemb_p = jnp.full((128,), -1e30, f32).at[:100].set(bemb)
    emb8 = _emb_kernel(p8, wemb8, jnp.tile(bemb_p, 8).reshape(1, 1024))

    pool = _pool_kernel(emb8.reshape(N_PAD, 128), batch2d, zeros_pool)

    out_pad = _head_kernel(
        pool,
        _pad2(Wm0, 128, 64), _row(bm0, 64), _row(gm0 * _BN_SCALE, 64),
        _row(bem0, 64),
        _pad2(Wm1, 64, 32), _row(bm1, 32), _row(gm1 * _BN_SCALE, 32),
        _row(bem1, 32),
        _pad2(Wout, 32, 128), _row(bout, 128))
    return out_pad[:, :4]


# speedup vs baseline: 12.4662x; 1.0491x over previous
import functools

import jax
import jax.numpy as jnp
from jax import lax
from jax.experimental import pallas as pl
from jax.experimental.pallas import tpu as pltpu
from jax.experimental.pallas import tpu_sc as plsc

N = 100000
E = 1600000
NUM_GRAPHS = 512

NC = 2
NS = 16
L = 16

N_PAD = 100352
STRIPE = N_PAD // NS
BN = 2048
N_BLOCKS = N_PAD // BN

K_BATCH = 4
STEP_E = K_BATCH * 128
E_TOT = E + N
E_PAD = 1703936
ROWS_PER_TILE = (E_PAD // NS) // 128
AGG_STEPS = ROWS_PER_TILE // K_BATCH
DEG_STEPS = (E_PAD // (NC * NS)) // STEP_E

POOL_ROWS = 640
POOL_STEP = 128
POOL_BLOCKS = N_PAD // POOL_STEP
POOL_STEPS = -(-POOL_BLOCKS // (NC * NS))

_MESH = plsc.VectorSubcoreMesh(
    core_axis_name="c", subcore_axis_name="s", num_cores=NC, num_subcores=NS)


def _pad2(a, r, c):
    return jnp.pad(a, ((0, r - a.shape[0]), (0, c - a.shape[1])))


def _row(a, c):
    return jnp.pad(a, (0, c - a.shape[0])).reshape(1, c)


def _kron8(w):
    k, d = w.shape
    out = jnp.zeros((8, k, 8, d), w.dtype)
    out = out.at[jnp.arange(8), :, jnp.arange(8), :].set(
        jnp.broadcast_to(w, (8, k, d)))
    return out.reshape(8 * k, 8 * d)


def _deg_body(dst2d, zeros16, out, acc, dstb, ones_buf, isem, ssem):
    cid = lax.axis_index("c")
    sid = lax.axis_index("s")
    one_row = jnp.full((L,), 1.0, jnp.float32)
    for r in range(128):
        ones_buf[r, :] = one_row
    pltpu.sync_copy(zeros16.at[pl.ds(sid * STRIPE, STRIPE)],
                    acc.at[pl.ds(sid * STRIPE, STRIPE)])
    plsc.subcore_barrier()

    base_rows = (cid * NS + sid) * (DEG_STEPS * K_BATCH)

    def issue_idx(step, slot):
        pltpu.async_copy(dst2d.at[pl.ds(base_rows + step * K_BATCH, K_BATCH)],
                         dstb.at[pl.ds(slot * K_BATCH, K_BATCH)], isem)

    def wait_idx(slot):
        pltpu.make_async_copy(dst2d.at[pl.ds(0, K_BATCH)],
                              dstb.at[pl.ds(slot * K_BATCH, K_BATCH)],
                              isem).wait()

    def fire_scatter(slot):
        for b in range(K_BATCH):
            pltpu.async_copy(ones_buf, acc.at[dstb.at[slot * K_BATCH + b]],
                             ssem, add=True)

    def drain_scatter():
        for b in range(K_BATCH):
            pltpu.make_async_copy(ones_buf, acc.at[pl.ds(0, 128)],
                                  ssem).wait()

    issue_idx(0, 0)

    def step2(j2, carry):
        s0 = 2 * j2

        @pl.when(j2 > 0)
        def _():
            drain_scatter()
        issue_idx(s0 + 1, 1)
        wait_idx(0)
        fire_scatter(0)
        drain_scatter()

        @pl.when(j2 < DEG_STEPS // 2 - 1)
        def _():
            issue_idx(s0 + 2, 0)
        wait_idx(1)
        fire_scatter(1)
        return carry

    lax.fori_loop(0, DEG_STEPS // 2, step2, 0)
    drain_scatter()
    plsc.subcore_barrier()
    pltpu.sync_copy(acc.at[pl.ds(sid * STRIPE, STRIPE)],
                    out.at[cid, pl.ds(sid * STRIPE, STRIPE)])


def _deg_kernel(dst2d, zeros16):
    return pl.kernel(
        _deg_body,
        out_type=jax.ShapeDtypeStruct((NC, N_PAD, L), jnp.float32),
        mesh=_MESH,
        compiler_params=pltpu.CompilerParams(use_tc_tiling_on_sc=False),
        scratch_types=[
            pltpu.VMEM_SHARED((N_PAD, L), jnp.float32),
            pltpu.VMEM((2 * K_BATCH, 128), jnp.int32),
            pltpu.VMEM((128, L), jnp.float32),
            pltpu.SemaphoreType.DMA,
            pltpu.SemaphoreType.DMA,
        ],
    )(dst2d, zeros16)


def _agg_body(nch, src2d, dst2d, p2d, zeros16, out, acc, srcb, dstb, gidxb,
              rows, isem, gsem, ssem):
    cid = lax.axis_index("c")
    sid = lax.axis_index("s")

    def wait_idx(slot):
        for ref in (srcb, dstb):
            pltpu.make_async_copy(
                src2d.at[pl.ds(0, K_BATCH)],
                ref.at[pl.ds(slot * K_BATCH, K_BATCH)], isem).wait()

    def compute_gidx(chunk, slot):
        for v in range(K_BATCH):
            r = slot * K_BATCH + v
            for u in range(128 // L):
                s16 = srcb[r, pl.ds(u * L, L)]
                gidxb[r, pl.ds(u * L, L)] = s16 * nch + chunk

    def fire_gather(slot):
        for b in range(K_BATCH):
            pltpu.async_copy(
                p2d.at[gidxb.at[slot * K_BATCH + b]],
                rows.at[pl.ds((slot * K_BATCH + b) * 128, 128)], gsem)

    def drain_gather(slot):
        for b in range(K_BATCH):
            pltpu.make_async_copy(
                p2d.at[pl.ds(0, 128)],
                rows.at[pl.ds((slot * K_BATCH + b) * 128, 128)], gsem).wait()

    def fire_scatter(slot):
        for b in range(K_BATCH):
            pltpu.async_copy(
                rows.at[pl.ds((slot * K_BATCH + b) * 128, 128)],
                acc.at[dstb.at[slot * K_BATCH + b]], ssem, add=True)

    def drain_scatter(slot):
        for b in range(K_BATCH):
            pltpu.make_async_copy(
                rows.at[pl.ds((slot * K_BATCH + b) * 128, 128)],
                acc.at[pl.ds(0, 128)], ssem).wait()

    def run_pass(chunk, slab, base0, steps):
        pltpu.sync_copy(zeros16.at[pl.ds(sid * STRIPE, STRIPE)],
                        acc.at[pl.ds(sid * STRIPE, STRIPE)])
        plsc.subcore_barrier()

        def idx_base(step):
            return base0 + step * K_BATCH

        def issue2(step, slot):
            pltpu.async_copy(src2d.at[pl.ds(idx_base(step), K_BATCH)],
                             srcb.at[pl.ds(slot * K_BATCH, K_BATCH)], isem)
            pltpu.async_copy(dst2d.at[pl.ds(idx_base(step), K_BATCH)],
                             dstb.at[pl.ds(slot * K_BATCH, K_BATCH)], isem)

        pltpu.sync_copy(src2d.at[pl.ds(idx_base(0), K_BATCH)],
                        srcb.at[pl.ds(0, K_BATCH)])
        compute_gidx(chunk, 0)
        fire_gather(0)
        pltpu.sync_copy(dst2d.at[pl.ds(idx_base(0), K_BATCH)],
                        dstb.at[pl.ds(0, K_BATCH)])

        def step2(j2, carry2):
            s0 = 2 * j2
            @pl.when(j2 > 0)
            def _():
                drain_scatter(1)
            issue2(s0 + 1, 1)
            drain_gather(0)
            fire_scatter(0)
            wait_idx(1)
            compute_gidx(chunk, 1)
            fire_gather(1)
            drain_scatter(0)

            @pl.when(j2 < steps // 2 - 1)
            def _():
                issue2(s0 + 2, 0)
            drain_gather(1)
            fire_scatter(1)

            @pl.when(j2 < steps // 2 - 1)
            def _():
                wait_idx(0)
                compute_gidx(chunk, 0)
                fire_gather(0)
            return carry2

        lax.fori_loop(0, steps // 2, step2, 0)
        drain_scatter(1)
        plsc.subcore_barrier()
        pltpu.sync_copy(
            acc.at[pl.ds(sid * STRIPE, STRIPE)],
            out.at[slab, pl.ds(sid * STRIPE, STRIPE)])
        plsc.subcore_barrier()

    full = nch // 2
    if nch % 2 == 0:
        def chunk_body(ci, carry):
            chunk = cid + 2 * ci
            run_pass(chunk, chunk, sid * ROWS_PER_TILE, AGG_STEPS)
            return carry

        lax.fori_loop(0, full, chunk_body, 0)
    else:
        def chunk_body(ci, carry):
            chunk = cid + 2 * ci
            run_pass(chunk, chunk, sid * ROWS_PER_TILE, AGG_STEPS)
            return carry

        lax.fori_loop(0, (nch - 1) // 2, chunk_body, 0)
        half_rows = (E_PAD // 128) // 2
        run_pass(nch - 1, (nch - 1) + cid,
                 cid * half_rows + sid * (ROWS_PER_TILE // 2),
                 AGG_STEPS // 2)


@functools.lru_cache(maxsize=None)
def _agg_kernel(nch):
    nslab = nch + (nch % 2)
    return pl.kernel(
        functools.partial(_agg_body, nch),
        out_type=jax.ShapeDtypeStruct((nslab, N_PAD, L), jnp.float32),
        mesh=_MESH,
        compiler_params=pltpu.CompilerParams(use_tc_tiling_on_sc=False),
        scratch_types=[
            pltpu.VMEM_SHARED((N_PAD, L), jnp.float32),
            pltpu.VMEM((2 * K_BATCH, 128), jnp.int32),
            pltpu.VMEM((2 * K_BATCH, 128), jnp.int32),
            pltpu.VMEM((2 * K_BATCH, 128), jnp.int32),
            pltpu.VMEM((2 * STEP_E, L), jnp.float32),
            pltpu.SemaphoreType.DMA,
            pltpu.SemaphoreType.DMA,
            pltpu.SemaphoreType.DMA,
        ],
    )


def _pool_body(emb, batch2d, zeros_pool, out, acc, rbuf, bidx):
    cid = lax.axis_index("c")
    sid = lax.axis_index("s")

    @pl.when(sid == 0)
    def _z():
        pltpu.sync_copy(zeros_pool, acc)

    plsc.subcore_barrier()
    wid = cid * NS + sid

    def step(j, carry):
        blk = wid + (NC * NS) * j

        @pl.when(blk < POOL_BLOCKS)
        def _():
            rowbase = blk * POOL_STEP
            pltpu.sync_copy(emb.at[pl.ds(rowbase, POOL_STEP)], rbuf)
            pltpu.sync_copy(batch2d.at[pl.ds(blk, 1)], bidx)
            pltpu.sync_copy(rbuf, acc.at[bidx.at[0]], add=True)

        return carry

    lax.fori_loop(0, POOL_STEPS, step, 0)
    plsc.subcore_barrier()
    rows_per_tile = POOL_ROWS // NS
    pltpu.sync_copy(acc.at[pl.ds(sid * rows_per_tile, rows_per_tile)],
                    out.at[cid, pl.ds(sid * rows_per_tile, rows_per_tile)])


def _pool_kernel(emb, batch2d, zeros_pool):
    return pl.kernel(
        _pool_body,
        out_type=jax.ShapeDtypeStruct((NC, POOL_ROWS, 128), jnp.float32),
        mesh=_MESH,
        compiler_params=pltpu.CompilerParams(use_tc_tiling_on_sc=False),
        scratch_types=[
            pltpu.VMEM_SHARED((POOL_ROWS, 128), jnp.float32),
            pltpu.VMEM((POOL_STEP, 128), jnp.float32),
            pltpu.VMEM((1, POOL_STEP), jnp.int32),
        ],
    )(emb, batch2d, zeros_pool)


def _prep_body(deg8_ref, x8_ref, b48_ref, dinv8_ref, p08_ref):
    deg = deg8_ref[0] + deg8_ref[1]
    dinv8 = jnp.where(deg > 0, lax.rsqrt(deg), 0.0)
    dinv8_ref[...] = dinv8
    dv48 = jnp.dot(dinv8, b48_ref[...], preferred_element_type=jnp.float32)
    p08_ref[...] = x8_ref[...] * dv48


def _prep_kernel(deg8, x8):
    return pl.pallas_call(
        _prep_body,
        grid=(N_BLOCKS,),
        in_specs=[
            pl.BlockSpec((NC, BN // 8, 128), lambda i: (0, i, 0)),
            pl.BlockSpec((BN // 8, 384), lambda i: (i, 0)),
            pl.BlockSpec((128, 384), lambda i: (0, 0)),
        ],
        out_specs=[
            pl.BlockSpec((BN // 8, 128), lambda i: (i, 0)),
            pl.BlockSpec((BN // 8, 384), lambda i: (i, 0)),
        ],
        out_shape=[
            jax.ShapeDtypeStruct((N_PAD // 8, 128), jnp.float32),
            jax.ShapeDtypeStruct((N_PAD // 8, 384), jnp.float32),
        ],
    )(deg8, x8, _kron8(jnp.full((L, 48), 1.0 / L, jnp.float32)))


def _layer_body(final, nch, u8_ref, dinv8_ref, w8_ref, bd_ref, b8_ref,
                gs8_ref, be8_ref, out_ref, zs, dv):
    c = pl.program_id(1)

    @pl.when(c == 0)
    def _init():
        zs[...] = jnp.broadcast_to(b8_ref[...], zs.shape)
        if not final:
            dv[...] = jnp.dot(dinv8_ref[...], bd_ref[...],
                              preferred_element_type=jnp.float32)

    wc = w8_ref[pl.ds(c, 1)][0]
    zs[...] += jnp.dot(u8_ref[0] * dinv8_ref[...], wc,
                       preferred_element_type=jnp.float32)

    @pl.when(c == nch - 1)
    def _fin():
        h = jnp.maximum(zs[...], 0.0) * gs8_ref[...] + be8_ref[...]
        if not final:
            h = h * dv[...]
        out_ref[...] = h


def _layer_kernel(u8, dinv8, w8, bd, b8, gs8, be8, final):
    nch = u8.shape[0]
    dout8 = w8.shape[2]
    return pl.pallas_call(
        functools.partial(_layer_body, final, nch),
        grid=(N_BLOCKS, nch),
        in_specs=[
            pl.BlockSpec((1, BN // 8, 128), lambda i, c: (c, i, 0)),
            pl.BlockSpec((BN // 8, 128), lambda i, c: (i, 0)),
            pl.BlockSpec((nch, 128, dout8), lambda i, c: (0, 0, 0)),
            pl.BlockSpec((128, dout8), lambda i, c: (0, 0)),
            pl.BlockSpec((1, dout8), lambda i, c: (0, 0)),
            pl.BlockSpec((1, dout8), lambda i, c: (0, 0)),
            pl.BlockSpec((1, dout8), lambda i, c: (0, 0)),
        ],
        out_specs=pl.BlockSpec((BN // 8, dout8), lambda i, c: (i, 0)),
        out_shape=jax.ShapeDtypeStruct((N_PAD // 8, dout8), jnp.float32),
        scratch_shapes=[pltpu.VMEM((BN // 8, dout8), jnp.float32),
                        pltpu.VMEM((BN // 8, dout8), jnp.float32)],
    )(u8, dinv8, w8, bd, b8, gs8, be8)


def _emb_body(h8_ref, w8_ref, b8_ref, out_ref):
    z8 = jnp.dot(h8_ref[...], w8_ref[...],
                 preferred_element_type=jnp.float32) + b8_ref[...]
    for k in range(8):
        zk = z8[:, k * 128:(k + 1) * 128]
        m = jnp.max(zk, axis=-1, keepdims=True)
        e = jnp.exp(zk - m)
        out_ref[:, k * 128:(k + 1) * 128] = e / jnp.sum(e, axis=-1,
                                                        keepdims=True)


def _emb_kernel(h8, w8, b8):
    din8 = h8.shape[1]
    return pl.pallas_call(
        _emb_body,
        grid=(N_BLOCKS,),
        in_specs=[
            pl.BlockSpec((BN // 8, din8), lambda i: (i, 0)),
            pl.BlockSpec((din8, 1024), lambda i: (0, 0)),
            pl.BlockSpec((1, 1024), lambda i: (0, 0)),
        ],
        out_specs=pl.BlockSpec((BN // 8, 1024), lambda i: (i, 0)),
        out_shape=jax.ShapeDtypeStruct((N_PAD // 8, 1024), jnp.float32),
    )(h8, w8, b8)


def _head_body(pool_ref, wm0, bm0, gsm0, bem0, wm1, bm1, gsm1, bem1, wo, bo,
               out_ref):
    hg = pool_ref[0, :NUM_GRAPHS, :] + pool_ref[1, :NUM_GRAPHS, :]
    z1 = jnp.dot(hg, wm0[...], preferred_element_type=jnp.float32) + bm0[...]
    h1 = jnp.maximum(z1, 0.0) * gsm0[...] + bem0[...]
    z2 = jnp.dot(h1, wm1[...], preferred_element_type=jnp.float32) + bm1[...]
    h2 = jnp.maximum(z2, 0.0) * gsm1[...] + bem1[...]
    out_ref[...] = jnp.dot(h2, wo[...],
                           preferred_element_type=jnp.float32) + bo[...]


def _head_kernel(pool, wm0, bm0, gsm0, bem0, wm1, bm1, gsm1, bem1, wo, bo):
    full = lambda a: pl.BlockSpec(a.shape, lambda: tuple(0 for _ in a.shape))
    return pl.pallas_call(
        _head_body,
        in_specs=[full(pool), full(wm0), full(bm0), full(gsm0), full(bem0),
                  full(wm1), full(bm1), full(gsm1), full(bem1), full(wo),
                  full(bo)],
        out_specs=pl.BlockSpec((NUM_GRAPHS, 128), lambda: (0, 0)),
        out_shape=jax.ShapeDtypeStruct((NUM_GRAPHS, 128), jnp.float32),
    )(pool, wm0, bm0, gsm0, bem0, wm1, bm1, gsm1, bem1, wo, bo)


_BN_SCALE = 1.0 / (1.0 + 1e-5) ** 0.5
_CONV_PADS = [(48, 64), (64, 80), (80, 112), (112, 128), (128, 160)]


def kernel(x, edge_index, batch, W0, b0, g0, be0, W1, b1, g1, be1, W2, b2, g2,
           be2, W3, b3, g3, be3, W4, b4, g4, be4, Wemb, bemb, Wm0, bm0, gm0,
           bem0, Wm1, bm1, gm1, bem1, Wout, bout):
    f32 = jnp.float32
    x_pad = _pad2(x, N_PAD, 48)
    loop = jnp.arange(N, dtype=jnp.int32)
    padidx = (N + (jnp.arange(E_PAD - E_TOT, dtype=jnp.int32) % 16))
    src2d = jnp.concatenate([edge_index[0], loop, padidx]).reshape(-1, 128)
    dst2d = jnp.concatenate([edge_index[1], loop, padidx]).reshape(-1, 128)
    batch2d = jnp.pad(batch, (0, N_PAD - N),
                      constant_values=NUM_GRAPHS).reshape(POOL_BLOCKS,
                                                          POOL_STEP)
    zeros16 = jnp.zeros((N_PAD, L), f32)
    zeros_pool = jnp.zeros((POOL_ROWS, 128), f32)

    convs = [(W0, b0, g0, be0), (W1, b1, g1, be1), (W2, b2, g2, be2),
             (W3, b3, g3, be3), (W4, b4, g4, be4)]

    deg16 = _deg_kernel(dst2d, zeros16)
    dinv8, p08 = _prep_kernel(deg16.reshape(NC, N_PAD // 8, 128),
                              x_pad.reshape(N_PAD // 8, 384))
    p2d = p08.reshape(N_PAD * 3, L)

    for li, ((din_p, dout_p), (W, b, g, be)) in enumerate(zip(_CONV_PADS,
                                                              convs)):
        nch = din_p // L
        chunks = list(range(nch)) + ([nch - 1] if nch % 2 else [])
        u = _agg_kernel(nch)(src2d, dst2d, p2d, zeros16)
        u8 = u.reshape(len(chunks), N_PAD // 8, 128)
        wp = _pad2(W, din_p, dout_p)
        w8 = jnp.stack([_kron8(wp[c * L:(c + 1) * L, :]) for c in chunks])
        p8 = _layer_kernel(
            u8, dinv8, w8,
            _kron8(jnp.full((L, dout_p), 1.0 / L, f32)),
            jnp.tile(_row(b, dout_p), (1, 8)),
            jnp.tile(_row(g * _BN_SCALE, dout_p), (1, 8)),
            jnp.tile(_row(be, dout_p), (1, 8)),
            final=(li == len(convs) - 1))
        p2d = p8.reshape(N_PAD * (dout_p // L), L)

    wemb8 = _kron8(_pad2(Wemb, 160, 128))
    bemb_p = jnp.full((128,), -1e30, f32).at[:100].set(bemb)
    emb8 = _emb_kernel(p8, wemb8, jnp.tile(bemb_p, 8).reshape(1, 1024))

    pool = _pool_kernel(emb8.reshape(N_PAD, 128), batch2d, zeros_pool)

    out_pad = _head_kernel(
        pool,
        _pad2(Wm0, 128, 64), _row(bm0, 64), _row(gm0 * _BN_SCALE, 64),
        _row(bem0, 64),
        _pad2(Wm1, 64, 32), _row(bm1, 32), _row(gm1 * _BN_SCALE, 32),
        _row(bem1, 32),
        _pad2(Wout, 32, 128), _row(bout, 128))
    return out_pad[:, :4]
